# Initial kernel scaffold; baseline (speedup 1.0000x reference)
import jax
import jax.numpy as jnp
from jax.experimental import pallas as pl


def kernel(x, edge_index, batch, Wl1, bl1, Wr1, Wla, bla, Wra, Wl2, bl2, Wr2, W3, b3, W4, b4):
    raise NotImplementedError("write your pallas kernel here")


# speedup vs baseline: 1.5448x; 1.5448x over previous
import functools

import jax
import jax.numpy as jnp
from jax import lax
from jax.experimental import pallas as pl
from jax.experimental.pallas import tpu as pltpu

N = 10000
E = 160000
B = 8
D = 256
C = 8

TN = 1000
TE = 2000

_I = False


def _dense_body(msg_ref, cnt_ref, x_ref, wl1_ref, wr1_ref, wla_ref, wra_ref,
                bl1_ref, bla_ref, z_ref, ssm_ref):
    cnt = jnp.maximum(cnt_ref[...], 1.0)
    a = msg_ref[...] / cnt
    x = x_ref[...]
    z = jnp.dot(a, wl1_ref[...], preferred_element_type=jnp.float32)
    z += jnp.dot(x, wr1_ref[...], preferred_element_type=jnp.float32)
    z += bl1_ref[...]
    z_ref[...] = jnp.maximum(z, 0.0)
    s = jnp.dot(a, wla_ref[...], preferred_element_type=jnp.float32)
    s += jnp.dot(x, wra_ref[...], preferred_element_type=jnp.float32)
    s += bla_ref[...]
    m = jnp.max(s, axis=1, keepdims=True)
    e = jnp.exp(s - m)
    ssm_ref[...] = e / jnp.sum(e, axis=1, keepdims=True)


def _dense_stage(msg, cnt2d, x, Wl1T, Wr1T, WlaT, WraT, bl1, bla):
    grid = (N // TN,)
    return pl.pallas_call(
        _dense_body,
        grid=grid,
        in_specs=[
            pl.BlockSpec((TN, D), lambda i: (i, 0)),
            pl.BlockSpec((TN, 1), lambda i: (i, 0)),
            pl.BlockSpec((TN, D), lambda i: (i, 0)),
            pl.BlockSpec((D, D), lambda i: (0, 0)),
            pl.BlockSpec((D, D), lambda i: (0, 0)),
            pl.BlockSpec((D, C), lambda i: (0, 0)),
            pl.BlockSpec((D, C), lambda i: (0, 0)),
            pl.BlockSpec((1, D), lambda i: (0, 0)),
            pl.BlockSpec((1, C), lambda i: (0, 0)),
        ],
        out_specs=[
            pl.BlockSpec((TN, D), lambda i: (i, 0)),
            pl.BlockSpec((TN, C), lambda i: (i, 0)),
        ],
        out_shape=[
            jax.ShapeDtypeStruct((N, D), jnp.float32),
            jax.ShapeDtypeStruct((N, C), jnp.float32),
        ],
        interpret=_I,
    )(msg, cnt2d, x, Wl1T, Wr1T, WlaT, WraT, bl1, bla)


def _pool_body(ssm_ref, z_ref, batch_ref, xp_ref, cnt_ref):
    i = pl.program_id(0)
    ids = lax.broadcasted_iota(jnp.int32, (1, B), 1)
    onehot = (batch_ref[...] == ids).astype(jnp.float32)
    ssm = ssm_ref[...]
    p = jnp.concatenate([ssm * onehot[:, b:b + 1] for b in range(B)], axis=1)
    xp = lax.dot_general(p, z_ref[...], (((0,), (0,)), ((), ())),
                         preferred_element_type=jnp.float32)
    cnts = jnp.sum(onehot, axis=0, keepdims=True)

    @pl.when(i == 0)
    def _():
        xp_ref[...] = xp
        cnt_ref[...] = cnts

    @pl.when(i != 0)
    def _():
        xp_ref[...] += xp
        cnt_ref[...] += cnts


def _pool_stage(ssm, z, batch2d):
    grid = (N // TN,)
    return pl.pallas_call(
        _pool_body,
        grid=grid,
        in_specs=[
            pl.BlockSpec((TN, C), lambda i: (i, 0)),
            pl.BlockSpec((TN, D), lambda i: (i, 0)),
            pl.BlockSpec((TN, 1), lambda i: (i, 0)),
        ],
        out_specs=[
            pl.BlockSpec((B * C, D), lambda i: (0, 0)),
            pl.BlockSpec((1, B), lambda i: (0, 0)),
        ],
        out_shape=[
            jax.ShapeDtypeStruct((B * C, D), jnp.float32),
            jax.ShapeDtypeStruct((1, B), jnp.float32),
        ],
        interpret=_I,
    )(ssm, z, batch2d)


def _adj_body(src_ref, dst_ref, u_ref, v_ref, cnt_ref, a_ref):
    i = pl.program_id(0)
    cnts = cnt_ref[...]
    rows = lax.broadcasted_iota(jnp.int32, (B, B), 0)
    cols = lax.broadcasted_iota(jnp.int32, (B, B), 1)
    tri = (rows <= cols).astype(jnp.float32)
    hi_f = jnp.dot(cnts, tri, preferred_element_type=jnp.float32)
    hi = hi_f.astype(jnp.int32)
    lo = hi - cnts.astype(jnp.int32)
    src = src_ref[...]
    dst = dst_ref[...]
    oh_s = (src >= lo) & (src < hi)
    oh_d = (dst >= lo) & (dst < hi)
    m8 = (oh_s & oh_d).astype(jnp.float32)
    u = u_ref[...]
    p = jnp.concatenate([u * m8[:, b:b + 1] for b in range(B)], axis=1)
    a = lax.dot_general(p, v_ref[...], (((0,), (0,)), ((), ())),
                        preferred_element_type=jnp.float32)

    @pl.when(i == 0)
    def _():
        a_ref[...] = a

    @pl.when(i != 0)
    def _():
        a_ref[...] += a


def _adj_stage(src2d, dst2d, u, v, counts):
    grid = (E // TE,)
    return pl.pallas_call(
        _adj_body,
        grid=grid,
        in_specs=[
            pl.BlockSpec((TE, 1), lambda i: (i, 0)),
            pl.BlockSpec((TE, 1), lambda i: (i, 0)),
            pl.BlockSpec((TE, C), lambda i: (i, 0)),
            pl.BlockSpec((TE, C), lambda i: (i, 0)),
            pl.BlockSpec((1, B), lambda i: (0, 0)),
        ],
        out_specs=pl.BlockSpec((B * C, C), lambda i: (0, 0)),
        out_shape=jax.ShapeDtypeStruct((B * C, C), jnp.float32),
        interpret=_I,
    )(src2d, dst2d, u, v, counts)


def _head1_body(a_ref, xp_ref, wl2_ref, wr2_ref, bl2_ref, zp_ref):
    xp = xp_ref[...]
    ones = jnp.ones((C, 1), jnp.float32)
    parts = []
    for b in range(B):
        ab = a_ref[b * C:(b + 1) * C, :]
        mf = (ab != 0.0).astype(jnp.float32)
        c2 = lax.dot_general(mf, ones, (((0,), (0,)), ((), ())),
                             preferred_element_type=jnp.float32)
        c2 = jnp.maximum(c2, 1.0)
        xb = xp[b * C:(b + 1) * C, :]
        ag = lax.dot_general(mf, xb, (((0,), (0,)), ((), ())),
                             preferred_element_type=jnp.float32)
        parts.append(ag / c2)
    aggr2 = jnp.concatenate(parts, axis=0)
    zp = jnp.dot(aggr2, wl2_ref[...], preferred_element_type=jnp.float32)
    zp += jnp.dot(xp, wr2_ref[...], preferred_element_type=jnp.float32)
    zp += bl2_ref[...]
    zp_ref[...] = jnp.maximum(zp, 0.0)


def _head1_stage(a_stack, x_pool, Wl2T, Wr2T, bl2):
    return pl.pallas_call(
        _head1_body,
        out_shape=jax.ShapeDtypeStruct((B * C, D), jnp.float32),
        interpret=_I,
    )(a_stack, x_pool, Wl2T, Wr2T, bl2)


def _head2_body(ge_ref, w3_ref, b3_ref, w4_ref, b4_ref, o_ref):
    h = jnp.dot(ge_ref[...], w3_ref[...], preferred_element_type=jnp.float32)
    h = jnp.maximum(h + b3_ref[...], 0.0)
    o = jnp.dot(h, w4_ref[...], preferred_element_type=jnp.float32)
    o_ref[...] = o + b4_ref[...]


def _head2_stage(ge, W3T, b3, W4T, b4):
    return pl.pallas_call(
        _head2_body,
        out_shape=jax.ShapeDtypeStruct((B, 1), jnp.float32),
        interpret=_I,
    )(ge, W3T, b3, W4T, b4)


def kernel(x, edge_index, batch, Wl1, bl1, Wr1, Wla, bla, Wra, Wl2, bl2, Wr2, W3, b3, W4, b4):
    src = edge_index[0]
    dst = edge_index[1]

    msg = jnp.zeros((N, D), jnp.float32).at[dst].add(x[src])
    cnt = jnp.zeros((N,), jnp.float32).at[dst].add(1.0)

    z, ssm = _dense_stage(
        msg, cnt.reshape(N, 1), x,
        Wl1.T, Wr1.T, Wla.T, Wra.T,
        bl1.reshape(1, D), bla.reshape(1, C))

    x_pool, counts = _pool_stage(ssm, z, batch.reshape(N, 1))

    u = ssm[src]
    v = ssm[dst]

    a_stack = _adj_stage(src.reshape(E, 1), dst.reshape(E, 1), u, v, counts)

    z_pool = _head1_stage(a_stack, x_pool, Wl2.T, Wr2.T, bl2.reshape(1, D))
    ge = z_pool.reshape(B, C * D)
    o = _head2_stage(ge, W3.T, b3.reshape(1, D), W4.T, b4.reshape(1, 1))
    return o.reshape(B)


# speedup vs baseline: 3.9186x; 2.5366x over previous
import functools

import jax
import jax.numpy as jnp
from jax import lax
from jax.experimental import pallas as pl
from jax.experimental.pallas import tpu as pltpu
from jax.experimental.pallas import tpu_sc as plsc

N = 10000
E = 160000
B = 8
D = 256
Q = 64
C = 8
CP = 16

TN = 1000
TE = 2000

NC = 2
NS = 16
EPS = E // NS
CHUNK = 400
K2 = 1000
RW = 1000

_I = False

def _sc_mesh():
    return plsc.VectorSubcoreMesh(core_axis_name="c", subcore_axis_name="s",
                                  num_cores=NC, num_subcores=NS)


def _sc_scatter_body(do_cnt, *refs):
    if do_cnt:
        (xab_hbm, idx2_hbm, dst_hbm, zeros_hbm, zcnt_hbm, ones_hbm,
         msg_hbm, cnt_hbm,
         idx_s, idx_d, rows_v, ones_v, acc, acc_cnt, sem) = refs
    else:
        (xab_hbm, idx2_hbm, dst_hbm, zeros_hbm,
         msg_hbm,
         idx_s, idx_d, rows_v, acc, sem) = refs
    del refs
    core = lax.axis_index("c")
    sub = lax.axis_index("s")

    r0 = sub * RW

    @pl.when(sub < N // RW)
    def _():
        pltpu.sync_copy(zeros_hbm.at[pl.ds(r0, RW), :], acc.at[pl.ds(r0, RW), :])

    if do_cnt:
        @pl.when((core == 0) & (sub == 0))
        def _():
            pltpu.sync_copy(zcnt_hbm, acc_cnt)

        @pl.when(core == 0)
        def _():
            pltpu.sync_copy(ones_hbm, ones_v)

    plsc.subcore_barrier()

    base = sub * EPS
    for j in range(EPS // CHUNK):
        off = base + j * CHUNK
        pltpu.sync_copy(idx2_hbm.at[pl.ds(core * E + off, CHUNK)], idx_s)
        pltpu.sync_copy(dst_hbm.at[pl.ds(off, CHUNK)], idx_d)
        pltpu.async_copy(xab_hbm.at[idx_s], rows_v, sem).wait()
        pltpu.sync_copy(rows_v, acc.at[idx_d], add=True)

        if do_cnt:
            @pl.when(core == 0)
            def _():
                pltpu.sync_copy(ones_v, acc_cnt.at[idx_d], add=True)

    plsc.subcore_barrier()

    @pl.when(sub < N // RW)
    def _():
        pltpu.sync_copy(acc.at[pl.ds(r0, RW), :],
                        msg_hbm.at[pl.ds(core * N + r0, RW), :])

    if do_cnt:
        @pl.when((core == 0) & (sub == 0))
        def _():
            pltpu.sync_copy(acc_cnt, cnt_hbm)


def _sc_scatter_stage(x, src, dst):
    zeros = jnp.zeros((N, Q), jnp.float32)
    zcnt = jnp.zeros((N, CP), jnp.float32)
    ones = jnp.ones((CHUNK, CP), jnp.float32)
    idx2 = jnp.concatenate([src, src + N])
    base_scratch = [
        pltpu.VMEM((CHUNK,), jnp.int32),
        pltpu.VMEM((CHUNK,), jnp.int32),
        pltpu.VMEM((CHUNK, Q), jnp.float32),
    ]
    msg_t = jax.ShapeDtypeStruct((2 * N, Q), jnp.float32)
    f1 = pl.kernel(
        functools.partial(_sc_scatter_body, False),
        out_type=[msg_t],
        mesh=_sc_mesh(),
        compiler_params=pltpu.CompilerParams(use_tc_tiling_on_sc=False),
        scratch_types=base_scratch + [
            pltpu.VMEM_SHARED((N, Q), jnp.float32),
            pltpu.SemaphoreType.DMA,
        ],
    )
    f2 = pl.kernel(
        functools.partial(_sc_scatter_body, True),
        out_type=[msg_t, jax.ShapeDtypeStruct((N, CP), jnp.float32)],
        mesh=_sc_mesh(),
        compiler_params=pltpu.CompilerParams(use_tc_tiling_on_sc=False),
        scratch_types=base_scratch + [
            pltpu.VMEM((CHUNK, CP), jnp.float32),
            pltpu.VMEM_SHARED((N, Q), jnp.float32),
            pltpu.VMEM_SHARED((N, CP), jnp.float32),
            pltpu.SemaphoreType.DMA,
        ],
    )
    xab1 = jnp.concatenate([x[:, 0 * Q:1 * Q], x[:, 1 * Q:2 * Q]], axis=0)
    xab2 = jnp.concatenate([x[:, 2 * Q:3 * Q], x[:, 3 * Q:4 * Q]], axis=0)
    (m01,) = f1(xab1, idx2, dst, zeros)
    m23, cnt16 = f2(xab2, idx2, dst, zeros, zcnt, ones)
    return m01[:N], m01[N:], m23[:N], m23[N:], cnt16[:, :1]


def _sc_gather_body(ssm_hbm, se_hbm, uv_hbm, idx_v, rows_v, sem):
    core = lax.axis_index("c")
    sub = lax.axis_index("s")
    for j in range(EPS // K2):
        off = core * E + sub * EPS + j * K2
        pltpu.sync_copy(se_hbm.at[pl.ds(off, K2)], idx_v)
        pltpu.async_copy(ssm_hbm.at[idx_v], rows_v, sem).wait()
        pltpu.sync_copy(rows_v, uv_hbm.at[pl.ds(off, K2), :])


def _sc_gather_stage(ssm_pad, src, dst):
    se = jnp.concatenate([src, dst])
    f = pl.kernel(
        _sc_gather_body,
        out_type=jax.ShapeDtypeStruct((2 * E, CP), jnp.float32),
        mesh=_sc_mesh(),
        compiler_params=pltpu.CompilerParams(use_tc_tiling_on_sc=False),
        scratch_types=[
            pltpu.VMEM((K2,), jnp.int32),
            pltpu.VMEM((K2, CP), jnp.float32),
            pltpu.SemaphoreType.DMA,
        ],
    )
    uv = f(ssm_pad, se)
    return uv[:E], uv[E:]


def _dense_body(m0_ref, m1_ref, m2_ref, m3_ref, cnt_ref, x_ref, wl1_ref,
                wr1_ref, wla_ref, wra_ref, bl1_ref, bla_ref, z_ref, ssm_ref):
    cnt = jnp.maximum(cnt_ref[...], 1.0)
    a = jnp.concatenate(
        [m0_ref[...], m1_ref[...], m2_ref[...], m3_ref[...]], axis=1) / cnt
    x = x_ref[...]
    z = jnp.dot(a, wl1_ref[...], preferred_element_type=jnp.float32)
    z += jnp.dot(x, wr1_ref[...], preferred_element_type=jnp.float32)
    z += bl1_ref[...]
    z_ref[...] = jnp.maximum(z, 0.0)
    s = jnp.dot(a, wla_ref[...], preferred_element_type=jnp.float32)
    s += jnp.dot(x, wra_ref[...], preferred_element_type=jnp.float32)
    s += bla_ref[...]
    ids = lax.broadcasted_iota(jnp.int32, (1, CP), 1)
    s = jnp.where(ids < C, s, -1e30)
    m = jnp.max(s, axis=1, keepdims=True)
    e = jnp.exp(s - m)
    ssm_ref[...] = e / jnp.sum(e, axis=1, keepdims=True)


def _dense_stage(m0, m1, m2, m3, cnt2d, x, Wl1T, Wr1T, WlaT, WraT, bl1, bla):
    grid = (N // TN,)
    return pl.pallas_call(
        _dense_body,
        grid=grid,
        in_specs=[
            pl.BlockSpec((TN, Q), lambda i: (i, 0)),
            pl.BlockSpec((TN, Q), lambda i: (i, 0)),
            pl.BlockSpec((TN, Q), lambda i: (i, 0)),
            pl.BlockSpec((TN, Q), lambda i: (i, 0)),
            pl.BlockSpec((TN, 1), lambda i: (i, 0)),
            pl.BlockSpec((TN, D), lambda i: (i, 0)),
            pl.BlockSpec((D, D), lambda i: (0, 0)),
            pl.BlockSpec((D, D), lambda i: (0, 0)),
            pl.BlockSpec((D, CP), lambda i: (0, 0)),
            pl.BlockSpec((D, CP), lambda i: (0, 0)),
            pl.BlockSpec((1, D), lambda i: (0, 0)),
            pl.BlockSpec((1, CP), lambda i: (0, 0)),
        ],
        out_specs=[
            pl.BlockSpec((TN, D), lambda i: (i, 0)),
            pl.BlockSpec((TN, CP), lambda i: (i, 0)),
        ],
        out_shape=[
            jax.ShapeDtypeStruct((N, D), jnp.float32),
            jax.ShapeDtypeStruct((N, CP), jnp.float32),
        ],
        interpret=_I,
    )(m0, m1, m2, m3, cnt2d, x, Wl1T, Wr1T, WlaT, WraT, bl1, bla)


def _pool_body(ssm_ref, z_ref, batch_ref, xp_ref, cnt_ref):
    i = pl.program_id(0)
    ids = lax.broadcasted_iota(jnp.int32, (1, B), 1)
    onehot = (batch_ref[...] == ids).astype(jnp.float32)
    ssm = ssm_ref[...][:, :C]
    p = jnp.concatenate([ssm * onehot[:, b:b + 1] for b in range(B)], axis=1)
    xp = lax.dot_general(p, z_ref[...], (((0,), (0,)), ((), ())),
                         preferred_element_type=jnp.float32)
    cnts = jnp.sum(onehot, axis=0, keepdims=True)

    @pl.when(i == 0)
    def _():
        xp_ref[...] = xp
        cnt_ref[...] = cnts

    @pl.when(i != 0)
    def _():
        xp_ref[...] += xp
        cnt_ref[...] += cnts


def _pool_stage(ssm, z, batch2d):
    grid = (N // TN,)
    return pl.pallas_call(
        _pool_body,
        grid=grid,
        in_specs=[
            pl.BlockSpec((TN, CP), lambda i: (i, 0)),
            pl.BlockSpec((TN, D), lambda i: (i, 0)),
            pl.BlockSpec((TN, 1), lambda i: (i, 0)),
        ],
        out_specs=[
            pl.BlockSpec((B * C, D), lambda i: (0, 0)),
            pl.BlockSpec((1, B), lambda i: (0, 0)),
        ],
        out_shape=[
            jax.ShapeDtypeStruct((B * C, D), jnp.float32),
            jax.ShapeDtypeStruct((1, B), jnp.float32),
        ],
        interpret=_I,
    )(ssm, z, batch2d)


def _adj_body(src_ref, dst_ref, u_ref, v_ref, cnt_ref, a_ref):
    i = pl.program_id(0)
    cnts = cnt_ref[...]
    rows = lax.broadcasted_iota(jnp.int32, (B, B), 0)
    cols = lax.broadcasted_iota(jnp.int32, (B, B), 1)
    tri = (rows <= cols).astype(jnp.float32)
    hi_f = jnp.dot(cnts, tri, preferred_element_type=jnp.float32)
    hi = hi_f.astype(jnp.int32)
    lo = hi - cnts.astype(jnp.int32)
    src = src_ref[...]
    dst = dst_ref[...]
    oh_s = (src >= lo) & (src < hi)
    oh_d = (dst >= lo) & (dst < hi)
    m8 = (oh_s & oh_d).astype(jnp.float32)
    u = u_ref[...][:, :C]
    v = v_ref[...][:, :C]
    p = jnp.concatenate([u * m8[:, b:b + 1] for b in range(B)], axis=1)
    a = lax.dot_general(p, v, (((0,), (0,)), ((), ())),
                        preferred_element_type=jnp.float32)

    @pl.when(i == 0)
    def _():
        a_ref[...] = a

    @pl.when(i != 0)
    def _():
        a_ref[...] += a


def _adj_stage(src2d, dst2d, u, v, counts):
    grid = (E // TE,)
    return pl.pallas_call(
        _adj_body,
        grid=grid,
        in_specs=[
            pl.BlockSpec((TE, 1), lambda i: (i, 0)),
            pl.BlockSpec((TE, 1), lambda i: (i, 0)),
            pl.BlockSpec((TE, CP), lambda i: (i, 0)),
            pl.BlockSpec((TE, CP), lambda i: (i, 0)),
            pl.BlockSpec((1, B), lambda i: (0, 0)),
        ],
        out_specs=pl.BlockSpec((B * C, C), lambda i: (0, 0)),
        out_shape=jax.ShapeDtypeStruct((B * C, C), jnp.float32),
        interpret=_I,
    )(src2d, dst2d, u, v, counts)


def _head1_body(a_ref, xp_ref, wl2_ref, wr2_ref, bl2_ref, zp_ref):
    xp = xp_ref[...]
    ones = jnp.ones((C, 1), jnp.float32)
    parts = []
    for b in range(B):
        ab = a_ref[b * C:(b + 1) * C, :]
        mf = (ab != 0.0).astype(jnp.float32)
        c2 = lax.dot_general(mf, ones, (((0,), (0,)), ((), ())),
                             preferred_element_type=jnp.float32)
        c2 = jnp.maximum(c2, 1.0)
        xb = xp[b * C:(b + 1) * C, :]
        ag = lax.dot_general(mf, xb, (((0,), (0,)), ((), ())),
                             preferred_element_type=jnp.float32)
        parts.append(ag / c2)
    aggr2 = jnp.concatenate(parts, axis=0)
    zp = jnp.dot(aggr2, wl2_ref[...], preferred_element_type=jnp.float32)
    zp += jnp.dot(xp, wr2_ref[...], preferred_element_type=jnp.float32)
    zp += bl2_ref[...]
    zp_ref[...] = jnp.maximum(zp, 0.0)


def _head1_stage(a_stack, x_pool, Wl2T, Wr2T, bl2):
    return pl.pallas_call(
        _head1_body,
        out_shape=jax.ShapeDtypeStruct((B * C, D), jnp.float32),
        interpret=_I,
    )(a_stack, x_pool, Wl2T, Wr2T, bl2)


def _head2_body(ge_ref, w3_ref, b3_ref, w4_ref, b4_ref, o_ref):
    h = jnp.dot(ge_ref[...], w3_ref[...], preferred_element_type=jnp.float32)
    h = jnp.maximum(h + b3_ref[...], 0.0)
    o = jnp.dot(h, w4_ref[...], preferred_element_type=jnp.float32)
    o_ref[...] = o + b4_ref[...]


def _head2_stage(ge, W3T, b3, W4T, b4):
    return pl.pallas_call(
        _head2_body,
        out_shape=jax.ShapeDtypeStruct((B, 1), jnp.float32),
        interpret=_I,
    )(ge, W3T, b3, W4T, b4)


def kernel(x, edge_index, batch, Wl1, bl1, Wr1, Wla, bla, Wra, Wl2, bl2, Wr2, W3, b3, W4, b4):
    src = edge_index[0]
    dst = edge_index[1]

    m0, m1, m2, m3, cnt2d = _sc_scatter_stage(x, src, dst)

    WlaTp = jnp.pad(Wla.T, ((0, 0), (0, CP - C)))
    WraTp = jnp.pad(Wra.T, ((0, 0), (0, CP - C)))
    blap = jnp.pad(bla.reshape(1, C), ((0, 0), (0, CP - C)))
    z, ssm = _dense_stage(
        m0, m1, m2, m3, cnt2d, x,
        Wl1.T, Wr1.T, WlaTp, WraTp,
        bl1.reshape(1, D), blap)

    x_pool, counts = _pool_stage(ssm, z, batch.reshape(N, 1))

    u, v = _sc_gather_stage(ssm, src, dst)

    a_stack = _adj_stage(src.reshape(E, 1), dst.reshape(E, 1), u, v, counts)

    z_pool = _head1_stage(a_stack, x_pool, Wl2.T, Wr2.T, bl2.reshape(1, D))
    ge = z_pool.reshape(B, C * D)
    o = _head2_stage(ge, W3.T, b3.reshape(1, D), W4.T, b4.reshape(1, 1))
    return o.reshape(B)


# speedup vs baseline: 4.6655x; 1.1906x over previous
import functools

import jax
import jax.numpy as jnp
from jax import lax
from jax.experimental import pallas as pl
from jax.experimental.pallas import tpu as pltpu
from jax.experimental.pallas import tpu_sc as plsc

N = 10000
E = 160000
B = 8
D = 256
Q = 64
C = 8
CP = 16
CW = 8

TN = 1000
TE = 2000

NC = 2
NS = 16
EPS = E // NS
CHUNK = 1000
K2 = 2000
RW = 1000

_I = False


def _sc_mesh():
    return plsc.VectorSubcoreMesh(core_axis_name="c", subcore_axis_name="s",
                                  num_cores=NC, num_subcores=NS)


def _sc_scatter_body(xab1_hbm, xab2_hbm, idx2_hbm, dst_hbm, zeros_hbm,
                     zcnt_hbm, ones_hbm, msga_hbm, msgb_hbm, cnt_hbm,
                     idx_s, idx_d, rows_v, ones_v, acc, acc_cnt, sem):
    core = lax.axis_index("c")
    sub = lax.axis_index("s")

    r0 = sub * RW

    @pl.when(sub < N // RW)
    def _():
        pltpu.sync_copy(zeros_hbm.at[pl.ds(r0, RW), :], acc.at[pl.ds(r0, RW), :])

    @pl.when((core == 0) & (sub == 0))
    def _():
        pltpu.sync_copy(zcnt_hbm, acc_cnt)

    @pl.when(core == 0)
    def _():
        pltpu.sync_copy(ones_hbm, ones_v)

    plsc.subcore_barrier()

    base = sub * EPS
    for j in range(EPS // CHUNK):
        off = base + j * CHUNK
        pltpu.sync_copy(idx2_hbm.at[pl.ds(core * E + off, CHUNK)], idx_s)
        pltpu.sync_copy(dst_hbm.at[pl.ds(off, CHUNK)], idx_d)
        pltpu.async_copy(xab1_hbm.at[idx_s], rows_v, sem).wait()
        pltpu.sync_copy(rows_v, acc.at[idx_d], add=True)

        @pl.when(core == 0)
        def _():
            pltpu.sync_copy(ones_v, acc_cnt.at[idx_d], add=True)

    plsc.subcore_barrier()

    @pl.when(sub < N // RW)
    def _():
        pltpu.sync_copy(acc.at[pl.ds(r0, RW), :],
                        msga_hbm.at[pl.ds(core * N + r0, RW), :])
        pltpu.sync_copy(zeros_hbm.at[pl.ds(r0, RW), :], acc.at[pl.ds(r0, RW), :])

    @pl.when((core == 0) & (sub == 0))
    def _():
        pltpu.sync_copy(acc_cnt, cnt_hbm)

    plsc.subcore_barrier()

    for j in range(EPS // CHUNK):
        off = base + j * CHUNK
        pltpu.sync_copy(idx2_hbm.at[pl.ds(core * E + off, CHUNK)], idx_s)
        pltpu.sync_copy(dst_hbm.at[pl.ds(off, CHUNK)], idx_d)
        pltpu.async_copy(xab2_hbm.at[idx_s], rows_v, sem).wait()
        pltpu.sync_copy(rows_v, acc.at[idx_d], add=True)

    plsc.subcore_barrier()

    @pl.when(sub < N // RW)
    def _():
        pltpu.sync_copy(acc.at[pl.ds(r0, RW), :],
                        msgb_hbm.at[pl.ds(core * N + r0, RW), :])


def _sc_scatter_stage(x, src, dst):
    zeros = jnp.zeros((N, Q), jnp.float32)
    zcnt = jnp.zeros((N, CW), jnp.float32)
    ones = jnp.ones((CHUNK, CW), jnp.float32)
    idx2 = jnp.concatenate([src, src + N])
    msg_t = jax.ShapeDtypeStruct((2 * N, Q), jnp.float32)
    f = pl.kernel(
        _sc_scatter_body,
        out_type=[msg_t, msg_t, jax.ShapeDtypeStruct((N, CW), jnp.float32)],
        mesh=_sc_mesh(),
        compiler_params=pltpu.CompilerParams(use_tc_tiling_on_sc=False),
        scratch_types=[
            pltpu.VMEM((CHUNK,), jnp.int32),
            pltpu.VMEM((CHUNK,), jnp.int32),
            pltpu.VMEM((CHUNK, Q), jnp.float32),
            pltpu.VMEM((CHUNK, CW), jnp.float32),
            pltpu.VMEM_SHARED((N, Q), jnp.float32),
            pltpu.VMEM_SHARED((N, CW), jnp.float32),
            pltpu.SemaphoreType.DMA,
        ],
    )
    xab1 = jnp.concatenate([x[:, 0 * Q:1 * Q], x[:, 1 * Q:2 * Q]], axis=0)
    xab2 = jnp.concatenate([x[:, 2 * Q:3 * Q], x[:, 3 * Q:4 * Q]], axis=0)
    m01, m23, cnt16 = f(xab1, xab2, idx2, dst, zeros, zcnt, ones)
    return m01[:N], m01[N:], m23[:N], m23[N:], cnt16[:, :1]


def _sc_gather_body(ssm_hbm, se_hbm, uv_hbm, idx_v, rows_v, sem):
    core = lax.axis_index("c")
    sub = lax.axis_index("s")
    for j in range(EPS // K2):
        off = core * E + sub * EPS + j * K2
        pltpu.sync_copy(se_hbm.at[pl.ds(off, K2)], idx_v)
        pltpu.async_copy(ssm_hbm.at[idx_v], rows_v, sem).wait()
        pltpu.sync_copy(rows_v, uv_hbm.at[pl.ds(off, K2), :])


def _sc_gather_stage(ssm_pad, src, dst):
    se = jnp.concatenate([src, dst])
    f = pl.kernel(
        _sc_gather_body,
        out_type=jax.ShapeDtypeStruct((2 * E, CP), jnp.float32),
        mesh=_sc_mesh(),
        compiler_params=pltpu.CompilerParams(use_tc_tiling_on_sc=False),
        scratch_types=[
            pltpu.VMEM((K2,), jnp.int32),
            pltpu.VMEM((K2, CP), jnp.float32),
            pltpu.SemaphoreType.DMA,
        ],
    )
    return f(ssm_pad, se)


def _dense_body(m0_ref, m1_ref, m2_ref, m3_ref, cnt_ref, x_ref, batch_ref,
                wl1_ref, wr1_ref, wla_ref, wra_ref, bl1_ref, bla_ref,
                ssm_ref, xp_ref, gcnt_ref):
    i = pl.program_id(0)
    cnt = jnp.maximum(cnt_ref[...], 1.0)
    a = jnp.concatenate(
        [m0_ref[...], m1_ref[...], m2_ref[...], m3_ref[...]], axis=1) / cnt
    x = x_ref[...]
    z = jnp.dot(a, wl1_ref[...], preferred_element_type=jnp.float32)
    z += jnp.dot(x, wr1_ref[...], preferred_element_type=jnp.float32)
    z += bl1_ref[...]
    z = jnp.maximum(z, 0.0)
    s = jnp.dot(a, wla_ref[...], preferred_element_type=jnp.float32)
    s += jnp.dot(x, wra_ref[...], preferred_element_type=jnp.float32)
    s += bla_ref[...]
    ids = lax.broadcasted_iota(jnp.int32, (1, CP), 1)
    s = jnp.where(ids < C, s, -1e30)
    m = jnp.max(s, axis=1, keepdims=True)
    e = jnp.exp(s - m)
    ssm = e / jnp.sum(e, axis=1, keepdims=True)
    ssm_ref[...] = ssm

    gids = lax.broadcasted_iota(jnp.int32, (1, B), 1)
    onehot = (batch_ref[...] == gids).astype(jnp.float32)
    sc8 = ssm[:, :C]
    p = jnp.concatenate([sc8 * onehot[:, b:b + 1] for b in range(B)], axis=1)
    xp = lax.dot_general(p, z, (((0,), (0,)), ((), ())),
                         preferred_element_type=jnp.float32)
    cnts = jnp.sum(onehot, axis=0, keepdims=True)

    @pl.when(i == 0)
    def _():
        xp_ref[...] = xp
        gcnt_ref[...] = cnts

    @pl.when(i != 0)
    def _():
        xp_ref[...] += xp
        gcnt_ref[...] += cnts


def _dense_stage(m0, m1, m2, m3, cnt2d, x, batch2d,
                 Wl1T, Wr1T, WlaT, WraT, bl1, bla):
    grid = (N // TN,)
    return pl.pallas_call(
        _dense_body,
        grid=grid,
        in_specs=[
            pl.BlockSpec((TN, Q), lambda i: (i, 0)),
            pl.BlockSpec((TN, Q), lambda i: (i, 0)),
            pl.BlockSpec((TN, Q), lambda i: (i, 0)),
            pl.BlockSpec((TN, Q), lambda i: (i, 0)),
            pl.BlockSpec((TN, 1), lambda i: (i, 0)),
            pl.BlockSpec((TN, D), lambda i: (i, 0)),
            pl.BlockSpec((TN, 1), lambda i: (i, 0)),
            pl.BlockSpec((D, D), lambda i: (0, 0)),
            pl.BlockSpec((D, D), lambda i: (0, 0)),
            pl.BlockSpec((D, CP), lambda i: (0, 0)),
            pl.BlockSpec((D, CP), lambda i: (0, 0)),
            pl.BlockSpec((1, D), lambda i: (0, 0)),
            pl.BlockSpec((1, CP), lambda i: (0, 0)),
        ],
        out_specs=[
            pl.BlockSpec((TN, CP), lambda i: (i, 0)),
            pl.BlockSpec((B * C, D), lambda i: (0, 0)),
            pl.BlockSpec((1, B), lambda i: (0, 0)),
        ],
        out_shape=[
            jax.ShapeDtypeStruct((N, CP), jnp.float32),
            jax.ShapeDtypeStruct((B * C, D), jnp.float32),
            jax.ShapeDtypeStruct((1, B), jnp.float32),
        ],
        interpret=_I,
    )(m0, m1, m2, m3, cnt2d, x, batch2d, Wl1T, Wr1T, WlaT, WraT, bl1, bla)


def _adj_head_body(src_ref, dst_ref, u_ref, v_ref, gcnt_ref, xp_ref,
                   wl2_ref, wr2_ref, bl2_ref, w3_ref, b3_ref, w4_ref, b4_ref,
                   o_ref, a_acc):
    i = pl.program_id(0)
    cnts = gcnt_ref[...]
    rows = lax.broadcasted_iota(jnp.int32, (B, B), 0)
    cols = lax.broadcasted_iota(jnp.int32, (B, B), 1)
    tri = (rows <= cols).astype(jnp.float32)
    hi_f = jnp.dot(cnts, tri, preferred_element_type=jnp.float32)
    hi = hi_f.astype(jnp.int32)
    lo = hi - cnts.astype(jnp.int32)
    src = src_ref[...]
    dst = dst_ref[...]
    oh_s = (src >= lo) & (src < hi)
    oh_d = (dst >= lo) & (dst < hi)
    m8 = (oh_s & oh_d).astype(jnp.float32)
    u = u_ref[...][:, :C]
    v = v_ref[...][:, :C]
    p = jnp.concatenate([u * m8[:, b:b + 1] for b in range(B)], axis=1)
    a = lax.dot_general(p, v, (((0,), (0,)), ((), ())),
                        preferred_element_type=jnp.float32)

    @pl.when(i == 0)
    def _():
        a_acc[...] = a

    @pl.when(i != 0)
    def _():
        a_acc[...] += a

    @pl.when(i == pl.num_programs(0) - 1)
    def _():
        xp = xp_ref[...]
        ones = jnp.ones((C, 1), jnp.float32)
        parts = []
        for b in range(B):
            ab = a_acc[b * C:(b + 1) * C, :]
            mf = (ab != 0.0).astype(jnp.float32)
            c2 = lax.dot_general(mf, ones, (((0,), (0,)), ((), ())),
                                 preferred_element_type=jnp.float32)
            c2 = jnp.maximum(c2, 1.0)
            xb = xp[b * C:(b + 1) * C, :]
            ag = lax.dot_general(mf, xb, (((0,), (0,)), ((), ())),
                                 preferred_element_type=jnp.float32)
            parts.append(ag / c2)
        aggr2 = jnp.concatenate(parts, axis=0)
        zp = jnp.dot(aggr2, wl2_ref[...], preferred_element_type=jnp.float32)
        zp += jnp.dot(xp, wr2_ref[...], preferred_element_type=jnp.float32)
        zp += bl2_ref[...]
        zp = jnp.maximum(zp, 0.0)
        r64 = lax.broadcasted_iota(jnp.int32, (B * C, B * C), 0)
        c64 = lax.broadcasted_iota(jnp.int32, (B * C, B * C), 1)
        perm = ((r64 % B) * C + r64 // B == c64).astype(jnp.float32)
        zp_cm = jnp.dot(perm, zp, preferred_element_type=jnp.float32)
        ge = jnp.concatenate([zp_cm[c * B:(c + 1) * B, :] for c in range(C)],
                             axis=1)
        h = jnp.dot(ge, w3_ref[...], preferred_element_type=jnp.float32)
        h = jnp.maximum(h + b3_ref[...], 0.0)
        o = jnp.dot(h, w4_ref[...], preferred_element_type=jnp.float32)
        o_ref[...] = o + b4_ref[...]


def _adj_head_stage(src2d, dst2d, uv, counts, x_pool,
                    Wl2T, Wr2T, bl2, W3T, b3, W4T, b4):
    grid = (E // TE,)
    return pl.pallas_call(
        _adj_head_body,
        grid=grid,
        in_specs=[
            pl.BlockSpec((TE, 1), lambda i: (i, 0)),
            pl.BlockSpec((TE, 1), lambda i: (i, 0)),
            pl.BlockSpec((TE, CP), lambda i: (i, 0)),
            pl.BlockSpec((TE, CP), lambda i: (E // TE + i, 0)),
            pl.BlockSpec((1, B), lambda i: (0, 0)),
            pl.BlockSpec((B * C, D), lambda i: (0, 0)),
            pl.BlockSpec((D, D), lambda i: (0, 0)),
            pl.BlockSpec((D, D), lambda i: (0, 0)),
            pl.BlockSpec((1, D), lambda i: (0, 0)),
            pl.BlockSpec((C * D, D), lambda i: (0, 0)),
            pl.BlockSpec((1, D), lambda i: (0, 0)),
            pl.BlockSpec((D, 1), lambda i: (0, 0)),
            pl.BlockSpec((1, 1), lambda i: (0, 0)),
        ],
        out_specs=pl.BlockSpec((B, 1), lambda i: (0, 0)),
        out_shape=jax.ShapeDtypeStruct((B, 1), jnp.float32),
        scratch_shapes=[pltpu.VMEM((B * C, C), jnp.float32)],
        interpret=_I,
    )(src2d, dst2d, uv, uv, counts, x_pool,
      Wl2T, Wr2T, bl2, W3T, b3, W4T, b4)


def kernel(x, edge_index, batch, Wl1, bl1, Wr1, Wla, bla, Wra, Wl2, bl2, Wr2, W3, b3, W4, b4):
    src = edge_index[0]
    dst = edge_index[1]

    m0, m1, m2, m3, cnt2d = _sc_scatter_stage(x, src, dst)

    WlaTp = jnp.pad(Wla.T, ((0, 0), (0, CP - C)))
    WraTp = jnp.pad(Wra.T, ((0, 0), (0, CP - C)))
    blap = jnp.pad(bla.reshape(1, C), ((0, 0), (0, CP - C)))
    ssm, x_pool, counts = _dense_stage(
        m0, m1, m2, m3, cnt2d, x, batch.reshape(N, 1),
        Wl1.T, Wr1.T, WlaTp, WraTp,
        bl1.reshape(1, D), blap)

    uv = _sc_gather_stage(ssm, src, dst)

    o = _adj_head_stage(src.reshape(E, 1), dst.reshape(E, 1), uv, counts,
                        x_pool, Wl2.T, Wr2.T, bl2.reshape(1, D),
                        W3.T, b3.reshape(1, D), W4.T, b4.reshape(1, 1))
    return o.reshape(B)


# speedup vs baseline: 4.8195x; 1.0330x over previous
import functools

import jax
import jax.numpy as jnp
from jax import lax
from jax.experimental import pallas as pl
from jax.experimental.pallas import tpu as pltpu
from jax.experimental.pallas import tpu_sc as plsc

N = 10000
E = 160000
B = 8
D = 256
Q = 64
C = 8
CP = 16
CW = 8

TN = 1000
TE = 5000

NC = 2
NS = 16
EPS = E // NS
CHUNK = 400
K2 = 2000
RW = 1000

_I = False


def _sc_mesh():
    return plsc.VectorSubcoreMesh(core_axis_name="c", subcore_axis_name="s",
                                  num_cores=NC, num_subcores=NS)


def _sc_pass(xab_hbm, idx2_hbm, dst_hbm, idx_s, idx_d, rows_v, acc, acc_cnt,
             ones_v, sems, core, sub, do_cnt):
    base = sub * EPS
    nch = EPS // CHUNK

    pltpu.sync_copy(idx2_hbm.at[pl.ds(core * E + base, CHUNK)], idx_s.at[0])
    pltpu.sync_copy(dst_hbm.at[pl.ds(base, CHUNK)], idx_d.at[0])
    descs = [None, None]
    descs[0] = pltpu.async_copy(xab_hbm.at[idx_s.at[0]], rows_v.at[0], sems[0])
    for j in range(nch):
        b = j % 2
        nb = (j + 1) % 2
        if j + 1 < nch:
            off = base + (j + 1) * CHUNK
            pltpu.sync_copy(idx2_hbm.at[pl.ds(core * E + off, CHUNK)],
                            idx_s.at[nb])
            pltpu.sync_copy(dst_hbm.at[pl.ds(off, CHUNK)], idx_d.at[nb])
            descs[nb] = pltpu.async_copy(xab_hbm.at[idx_s.at[nb]],
                                         rows_v.at[nb], sems[nb])
        descs[b].wait()
        pltpu.sync_copy(rows_v.at[b], acc.at[idx_d.at[b]], add=True)
        if do_cnt:
            @pl.when(core == 0)
            def _():
                pltpu.sync_copy(ones_v, acc_cnt.at[idx_d.at[b]], add=True)


def _sc_scatter_body(xab1_hbm, xab2_hbm, idx2_hbm, dst_hbm, zeros_hbm,
                     zcnt_hbm, ones_hbm, msga_hbm, msgb_hbm, cnt_hbm,
                     idx_s, idx_d, rows_v, ones_v, acc, acc_cnt, sem0, sem1):
    core = lax.axis_index("c")
    sub = lax.axis_index("s")

    r0 = sub * RW

    @pl.when(sub < N // RW)
    def _():
        pltpu.sync_copy(zeros_hbm.at[pl.ds(r0, RW), :], acc.at[pl.ds(r0, RW), :])

    @pl.when((core == 0) & (sub == 0))
    def _():
        pltpu.sync_copy(zcnt_hbm, acc_cnt)

    @pl.when(core == 0)
    def _():
        pltpu.sync_copy(ones_hbm, ones_v)

    plsc.subcore_barrier()

    _sc_pass(xab1_hbm, idx2_hbm, dst_hbm, idx_s, idx_d, rows_v, acc, acc_cnt,
             ones_v, (sem0, sem1), core, sub, True)

    plsc.subcore_barrier()

    @pl.when(sub < N // RW)
    def _():
        pltpu.sync_copy(acc.at[pl.ds(r0, RW), :],
                        msga_hbm.at[pl.ds(core * N + r0, RW), :])
        pltpu.sync_copy(zeros_hbm.at[pl.ds(r0, RW), :], acc.at[pl.ds(r0, RW), :])

    @pl.when((core == 0) & (sub == 0))
    def _():
        pltpu.sync_copy(acc_cnt, cnt_hbm)

    plsc.subcore_barrier()

    _sc_pass(xab2_hbm, idx2_hbm, dst_hbm, idx_s, idx_d, rows_v, acc, acc_cnt,
             ones_v, (sem0, sem1), core, sub, False)

    plsc.subcore_barrier()

    @pl.when(sub < N // RW)
    def _():
        pltpu.sync_copy(acc.at[pl.ds(r0, RW), :],
                        msgb_hbm.at[pl.ds(core * N + r0, RW), :])


def _sc_scatter_stage(x, src, dst):
    zeros = jnp.zeros((N, Q), jnp.float32)
    zcnt = jnp.zeros((N, CW), jnp.float32)
    ones = jnp.ones((CHUNK, CW), jnp.float32)
    idx2 = jnp.concatenate([src, src + N])
    msg_t = jax.ShapeDtypeStruct((2 * N, Q), jnp.float32)
    f = pl.kernel(
        _sc_scatter_body,
        out_type=[msg_t, msg_t, jax.ShapeDtypeStruct((N, CW), jnp.float32)],
        mesh=_sc_mesh(),
        compiler_params=pltpu.CompilerParams(use_tc_tiling_on_sc=False),
        scratch_types=[
            pltpu.VMEM((2, CHUNK), jnp.int32),
            pltpu.VMEM((2, CHUNK), jnp.int32),
            pltpu.VMEM((2, CHUNK, Q), jnp.float32),
            pltpu.VMEM((CHUNK, CW), jnp.float32),
            pltpu.VMEM_SHARED((N, Q), jnp.float32),
            pltpu.VMEM_SHARED((N, CW), jnp.float32),
            pltpu.SemaphoreType.DMA,
            pltpu.SemaphoreType.DMA,
        ],
    )
    xab1 = jnp.concatenate([x[:, 0 * Q:1 * Q], x[:, 1 * Q:2 * Q]], axis=0)
    xab2 = jnp.concatenate([x[:, 2 * Q:3 * Q], x[:, 3 * Q:4 * Q]], axis=0)
    m01, m23, cnt16 = f(xab1, xab2, idx2, dst, zeros, zcnt, ones)
    return m01[:N], m01[N:], m23[:N], m23[N:], cnt16[:, :1]


def _sc_gather_body(ssm_hbm, se_hbm, uv_hbm, idx_v, rows_v, sem0, sem1):
    core = lax.axis_index("c")
    sub = lax.axis_index("s")
    base = core * E + sub * EPS
    nch = EPS // K2
    sems = (sem0, sem1)
    pltpu.sync_copy(se_hbm.at[pl.ds(base, K2)], idx_v.at[0])
    descs = [None, None]
    descs[0] = pltpu.async_copy(ssm_hbm.at[idx_v.at[0]], rows_v.at[0], sems[0])
    for j in range(nch):
        b = j % 2
        nb = (j + 1) % 2
        if j + 1 < nch:
            off = base + (j + 1) * K2
            pltpu.sync_copy(se_hbm.at[pl.ds(off, K2)], idx_v.at[nb])
            descs[nb] = pltpu.async_copy(ssm_hbm.at[idx_v.at[nb]],
                                         rows_v.at[nb], sems[nb])
        descs[b].wait()
        pltpu.sync_copy(rows_v.at[b], uv_hbm.at[pl.ds(base + j * K2, K2), :])


def _sc_gather_stage(ssm_pad, src, dst):
    se = jnp.concatenate([src, dst])
    f = pl.kernel(
        _sc_gather_body,
        out_type=jax.ShapeDtypeStruct((2 * E, CP), jnp.float32),
        mesh=_sc_mesh(),
        compiler_params=pltpu.CompilerParams(use_tc_tiling_on_sc=False),
        scratch_types=[
            pltpu.VMEM((2, K2), jnp.int32),
            pltpu.VMEM((2, K2, CP), jnp.float32),
            pltpu.SemaphoreType.DMA,
            pltpu.SemaphoreType.DMA,
        ],
    )
    return f(ssm_pad, se)


def _dense_body(m0_ref, m1_ref, m2_ref, m3_ref, cnt_ref, x_ref, batch_ref,
                wl1_ref, wr1_ref, wla_ref, wra_ref, bl1_ref, bla_ref,
                ssm_ref, xp_ref, gcnt_ref):
    i = pl.program_id(0)
    cnt = jnp.maximum(cnt_ref[...], 1.0)
    a = jnp.concatenate(
        [m0_ref[...], m1_ref[...], m2_ref[...], m3_ref[...]], axis=1) / cnt
    x = x_ref[...]
    z = jnp.dot(a, wl1_ref[...], preferred_element_type=jnp.float32)
    z += jnp.dot(x, wr1_ref[...], preferred_element_type=jnp.float32)
    z += bl1_ref[...]
    z = jnp.maximum(z, 0.0)
    s = jnp.dot(a, wla_ref[...], preferred_element_type=jnp.float32)
    s += jnp.dot(x, wra_ref[...], preferred_element_type=jnp.float32)
    s += bla_ref[...]
    ids = lax.broadcasted_iota(jnp.int32, (1, CP), 1)
    s = jnp.where(ids < C, s, -1e30)
    m = jnp.max(s, axis=1, keepdims=True)
    e = jnp.exp(s - m)
    ssm = e / jnp.sum(e, axis=1, keepdims=True)
    ssm_ref[...] = ssm

    gids = lax.broadcasted_iota(jnp.int32, (1, B), 1)
    onehot = (batch_ref[...] == gids).astype(jnp.float32)
    sc8 = ssm[:, :C]
    gid64 = lax.broadcasted_iota(jnp.int32, (1, B * C), 1) // C
    oh64 = (batch_ref[...] == gid64).astype(jnp.float32)
    p = jnp.concatenate([sc8] * B, axis=1) * oh64
    xp = lax.dot_general(p, z, (((0,), (0,)), ((), ())),
                         preferred_element_type=jnp.float32)
    cnts = jnp.sum(onehot, axis=0, keepdims=True)

    @pl.when(i == 0)
    def _():
        xp_ref[...] = xp
        gcnt_ref[...] = cnts

    @pl.when(i != 0)
    def _():
        xp_ref[...] += xp
        gcnt_ref[...] += cnts


def _dense_stage(m0, m1, m2, m3, cnt2d, x, batch2d,
                 Wl1T, Wr1T, WlaT, WraT, bl1, bla):
    grid = (N // TN,)
    return pl.pallas_call(
        _dense_body,
        grid=grid,
        in_specs=[
            pl.BlockSpec((TN, Q), lambda i: (i, 0)),
            pl.BlockSpec((TN, Q), lambda i: (i, 0)),
            pl.BlockSpec((TN, Q), lambda i: (i, 0)),
            pl.BlockSpec((TN, Q), lambda i: (i, 0)),
            pl.BlockSpec((TN, 1), lambda i: (i, 0)),
            pl.BlockSpec((TN, D), lambda i: (i, 0)),
            pl.BlockSpec((TN, 1), lambda i: (i, 0)),
            pl.BlockSpec((D, D), lambda i: (0, 0)),
            pl.BlockSpec((D, D), lambda i: (0, 0)),
            pl.BlockSpec((D, CP), lambda i: (0, 0)),
            pl.BlockSpec((D, CP), lambda i: (0, 0)),
            pl.BlockSpec((1, D), lambda i: (0, 0)),
            pl.BlockSpec((1, CP), lambda i: (0, 0)),
        ],
        out_specs=[
            pl.BlockSpec((TN, CP), lambda i: (i, 0)),
            pl.BlockSpec((B * C, D), lambda i: (0, 0)),
            pl.BlockSpec((1, B), lambda i: (0, 0)),
        ],
        out_shape=[
            jax.ShapeDtypeStruct((N, CP), jnp.float32),
            jax.ShapeDtypeStruct((B * C, D), jnp.float32),
            jax.ShapeDtypeStruct((1, B), jnp.float32),
        ],
        interpret=_I,
    )(m0, m1, m2, m3, cnt2d, x, batch2d, Wl1T, Wr1T, WlaT, WraT, bl1, bla)


def _adj_head_body(src_ref, dst_ref, u_ref, v_ref, gcnt_ref, xp_ref,
                   wl2_ref, wr2_ref, bl2_ref, w3_ref, b3_ref, w4_ref, b4_ref,
                   o_ref, a_acc):
    i = pl.program_id(0)
    cnts = gcnt_ref[...]
    rows8 = lax.broadcasted_iota(jnp.int32, (B, B * C), 0)
    cols8 = lax.broadcasted_iota(jnp.int32, (B, B * C), 1) // C
    tri64 = (rows8 <= cols8).astype(jnp.float32)
    hi64 = jnp.dot(cnts, tri64,
                   preferred_element_type=jnp.float32).astype(jnp.int32)
    cnt64 = jnp.dot(cnts, (rows8 == cols8).astype(jnp.float32),
                    preferred_element_type=jnp.float32).astype(jnp.int32)
    lo64 = hi64 - cnt64
    src = src_ref[...]
    dst = dst_ref[...]
    m64 = ((src >= lo64) & (src < hi64) & (dst >= lo64) & (dst < hi64))
    u = u_ref[...][:, :C]
    v = v_ref[...][:, :C]
    p = jnp.concatenate([u] * B, axis=1) * m64.astype(jnp.float32)
    a = lax.dot_general(p, v, (((0,), (0,)), ((), ())),
                        preferred_element_type=jnp.float32)

    @pl.when(i == 0)
    def _():
        a_acc[...] = a

    @pl.when(i != 0)
    def _():
        a_acc[...] += a

    @pl.when(i == pl.num_programs(0) - 1)
    def _():
        xp = xp_ref[...]
        ones = jnp.ones((C, 1), jnp.float32)
        parts = []
        for b in range(B):
            ab = a_acc[b * C:(b + 1) * C, :]
            mf = (ab != 0.0).astype(jnp.float32)
            c2 = lax.dot_general(mf, ones, (((0,), (0,)), ((), ())),
                                 preferred_element_type=jnp.float32)
            c2 = jnp.maximum(c2, 1.0)
            xb = xp[b * C:(b + 1) * C, :]
            ag = lax.dot_general(mf, xb, (((0,), (0,)), ((), ())),
                                 preferred_element_type=jnp.float32)
            parts.append(ag / c2)
        aggr2 = jnp.concatenate(parts, axis=0)
        zp = jnp.dot(aggr2, wl2_ref[...], preferred_element_type=jnp.float32)
        zp += jnp.dot(xp, wr2_ref[...], preferred_element_type=jnp.float32)
        zp += bl2_ref[...]
        zp = jnp.maximum(zp, 0.0)
        r64 = lax.broadcasted_iota(jnp.int32, (B * C, B * C), 0)
        c64 = lax.broadcasted_iota(jnp.int32, (B * C, B * C), 1)
        perm = ((r64 % B) * C + r64 // B == c64).astype(jnp.float32)
        zp_cm = jnp.dot(perm, zp, preferred_element_type=jnp.float32)
        ge = jnp.concatenate([zp_cm[c * B:(c + 1) * B, :] for c in range(C)],
                             axis=1)
        h = jnp.dot(ge, w3_ref[...], preferred_element_type=jnp.float32)
        h = jnp.maximum(h + b3_ref[...], 0.0)
        o = jnp.dot(h, w4_ref[...], preferred_element_type=jnp.float32)
        o_ref[...] = o + b4_ref[...]


def _adj_head_stage(src2d, dst2d, uv, counts, x_pool,
                    Wl2T, Wr2T, bl2, W3T, b3, W4T, b4):
    grid = (E // TE,)
    return pl.pallas_call(
        _adj_head_body,
        grid=grid,
        in_specs=[
            pl.BlockSpec((TE, 1), lambda i: (i, 0)),
            pl.BlockSpec((TE, 1), lambda i: (i, 0)),
            pl.BlockSpec((TE, CP), lambda i: (i, 0)),
            pl.BlockSpec((TE, CP), lambda i: (E // TE + i, 0)),
            pl.BlockSpec((1, B), lambda i: (0, 0)),
            pl.BlockSpec((B * C, D), lambda i: (0, 0)),
            pl.BlockSpec((D, D), lambda i: (0, 0)),
            pl.BlockSpec((D, D), lambda i: (0, 0)),
            pl.BlockSpec((1, D), lambda i: (0, 0)),
            pl.BlockSpec((C * D, D), lambda i: (0, 0)),
            pl.BlockSpec((1, D), lambda i: (0, 0)),
            pl.BlockSpec((D, 1), lambda i: (0, 0)),
            pl.BlockSpec((1, 1), lambda i: (0, 0)),
        ],
        out_specs=pl.BlockSpec((B, 1), lambda i: (0, 0)),
        out_shape=jax.ShapeDtypeStruct((B, 1), jnp.float32),
        scratch_shapes=[pltpu.VMEM((B * C, C), jnp.float32)],
        interpret=_I,
    )(src2d, dst2d, uv, uv, counts, x_pool,
      Wl2T, Wr2T, bl2, W3T, b3, W4T, b4)


def kernel(x, edge_index, batch, Wl1, bl1, Wr1, Wla, bla, Wra, Wl2, bl2, Wr2, W3, b3, W4, b4):
    src = edge_index[0]
    dst = edge_index[1]

    m0, m1, m2, m3, cnt2d = _sc_scatter_stage(x, src, dst)

    WlaTp = jnp.pad(Wla.T, ((0, 0), (0, CP - C)))
    WraTp = jnp.pad(Wra.T, ((0, 0), (0, CP - C)))
    blap = jnp.pad(bla.reshape(1, C), ((0, 0), (0, CP - C)))
    ssm, x_pool, counts = _dense_stage(
        m0, m1, m2, m3, cnt2d, x, batch.reshape(N, 1),
        Wl1.T, Wr1.T, WlaTp, WraTp,
        bl1.reshape(1, D), blap)

    uv = _sc_gather_stage(ssm, src, dst)

    o = _adj_head_stage(src.reshape(E, 1), dst.reshape(E, 1), uv, counts,
                        x_pool, Wl2.T, Wr2.T, bl2.reshape(1, D),
                        W3.T, b3.reshape(1, D), W4.T, b4.reshape(1, 1))
    return o.reshape(B)


# speedup vs baseline: 8.9667x; 1.8605x over previous
import functools

import jax
import jax.numpy as jnp
from jax import lax
from jax.experimental import pallas as pl
from jax.experimental.pallas import tpu as pltpu
from jax.experimental.pallas import tpu_sc as plsc

N = 10000
E = 160000
B = 8
D = 256
H = 128
Q = 64
C = 8
CP = 16
CW = 8
PK = 128 // CP

TN = 1000
TE = 8000

NC = 2
NS = 16
EPS = E // NS
CHUNK = 400
K2 = 2000
RW = 1000

_I = False


def _sc_mesh():
    return plsc.VectorSubcoreMesh(core_axis_name="c", subcore_axis_name="s",
                                  num_cores=NC, num_subcores=NS)


def _sc_pass(xh_hbm, idx2_hbm, dst_hbm, idx_s, idx_d, rows_v, acc, acc_cnt,
             ones_v, sems, core, sub, do_cnt):
    table = xh_hbm
    base = sub * EPS
    nch = EPS // CHUNK

    pltpu.sync_copy(idx2_hbm.at[pl.ds(core * E + base, CHUNK)], idx_s.at[0])
    pltpu.sync_copy(dst_hbm.at[pl.ds(base, CHUNK)], idx_d.at[0])
    descs = [None, None]
    descs[0] = pltpu.async_copy(table.at[idx_s.at[0]], rows_v.at[0], sems[0])
    for j in range(nch):
        b = j % 2
        nb = (j + 1) % 2
        if j + 1 < nch:
            off = base + (j + 1) * CHUNK
            pltpu.sync_copy(idx2_hbm.at[pl.ds(core * E + off, CHUNK)],
                            idx_s.at[nb])
            pltpu.sync_copy(dst_hbm.at[pl.ds(off, CHUNK)], idx_d.at[nb])
            descs[nb] = pltpu.async_copy(table.at[idx_s.at[nb]],
                                         rows_v.at[nb], sems[nb])
        descs[b].wait()
        pltpu.sync_copy(rows_v.at[b], acc.at[idx_d.at[b]], add=True)
        if do_cnt:
            @pl.when(core == 0)
            def _():
                pltpu.sync_copy(ones_v, acc_cnt.at[idx_d.at[b]], add=True)


def _sc_scatter_body(xl_hbm, xr_hbm, idx2_hbm, dst_hbm, zeros_hbm,
                     zcnt_hbm, ones_hbm, msgl_hbm, msgr_hbm, cnt_hbm,
                     idx_s, idx_d, rows_v, ones_v, acc, acc_cnt, sem0, sem1):
    core = lax.axis_index("c")
    sub = lax.axis_index("s")

    r0 = sub * RW

    @pl.when(sub < N // RW)
    def _():
        pltpu.sync_copy(zeros_hbm.at[pl.ds(r0, RW), :], acc.at[pl.ds(r0, RW), :])

    @pl.when((core == 0) & (sub == 0))
    def _():
        pltpu.sync_copy(zcnt_hbm, acc_cnt)

    @pl.when(core == 0)
    def _():
        pltpu.sync_copy(ones_hbm, ones_v)

    plsc.subcore_barrier()

    _sc_pass(xl_hbm, idx2_hbm, dst_hbm, idx_s, idx_d, rows_v, acc, acc_cnt,
             ones_v, (sem0, sem1), core, sub, True)

    plsc.subcore_barrier()

    @pl.when(sub < N // RW)
    def _():
        pltpu.sync_copy(acc.at[pl.ds(r0, RW), :],
                        msgl_hbm.at[pl.ds(r0, RW), pl.ds(core * Q, Q)])
        pltpu.sync_copy(zeros_hbm.at[pl.ds(r0, RW), :], acc.at[pl.ds(r0, RW), :])

    @pl.when((core == 0) & (sub == 0))
    def _():
        pltpu.sync_copy(acc_cnt, cnt_hbm)

    plsc.subcore_barrier()

    _sc_pass(xr_hbm, idx2_hbm, dst_hbm, idx_s, idx_d, rows_v, acc, acc_cnt,
             ones_v, (sem0, sem1), core, sub, False)

    plsc.subcore_barrier()

    @pl.when(sub < N // RW)
    def _():
        pltpu.sync_copy(acc.at[pl.ds(r0, RW), :],
                        msgr_hbm.at[pl.ds(r0, RW), pl.ds(core * Q, Q)])


def _sc_scatter_stage(x, src, dst):
    zeros = jnp.zeros((N, Q), jnp.float32)
    zcnt = jnp.zeros((N, CW), jnp.float32)
    ones = jnp.ones((CHUNK, CW), jnp.float32)
    idx2 = jnp.concatenate([2 * src, 2 * src + 1])
    msg_t = jax.ShapeDtypeStruct((N, H), jnp.float32)
    f = pl.kernel(
        _sc_scatter_body,
        out_type=[msg_t, msg_t, jax.ShapeDtypeStruct((N, CW), jnp.float32)],
        mesh=_sc_mesh(),
        compiler_params=pltpu.CompilerParams(use_tc_tiling_on_sc=False),
        scratch_types=[
            pltpu.VMEM((2, CHUNK), jnp.int32),
            pltpu.VMEM((2, CHUNK), jnp.int32),
            pltpu.VMEM((2, CHUNK, Q), jnp.float32),
            pltpu.VMEM((CHUNK, CW), jnp.float32),
            pltpu.VMEM_SHARED((N, Q), jnp.float32),
            pltpu.VMEM_SHARED((N, CW), jnp.float32),
            pltpu.SemaphoreType.DMA,
            pltpu.SemaphoreType.DMA,
        ],
    )
    xab1 = x[:, :H].reshape(2 * N, Q)
    xab2 = x[:, H:].reshape(2 * N, Q)
    msgl, msgr, cnt16 = f(xab1, xab2, idx2, dst, zeros, zcnt, ones)
    return msgl, msgr, cnt16[:, :1]


def _sc_gather_body(ssm_hbm, se_hbm, uv_hbm, idx_v, rows_v, sem0, sem1):
    core = lax.axis_index("c")
    sub = lax.axis_index("s")
    base = core * E + sub * EPS
    nch = EPS // K2
    sems = (sem0, sem1)
    pltpu.sync_copy(se_hbm.at[pl.ds(base, K2)], idx_v.at[0])
    descs = [None, None]
    descs[0] = pltpu.async_copy(ssm_hbm.at[idx_v.at[0]], rows_v.at[0], sems[0])
    for j in range(nch):
        b = j % 2
        nb = (j + 1) % 2
        if j + 1 < nch:
            off = base + (j + 1) * K2
            pltpu.sync_copy(se_hbm.at[pl.ds(off, K2)], idx_v.at[nb])
            descs[nb] = pltpu.async_copy(ssm_hbm.at[idx_v.at[nb]],
                                         rows_v.at[nb], sems[nb])
        descs[b].wait()
        pltpu.sync_copy(rows_v.at[b], uv_hbm.at[pl.ds(base + j * K2, K2), :])


def _sc_gather_stage(ssm_pad, src, dst):
    se = jnp.concatenate([src, dst])
    f = pl.kernel(
        _sc_gather_body,
        out_type=jax.ShapeDtypeStruct((2 * E, CP), jnp.float32),
        mesh=_sc_mesh(),
        compiler_params=pltpu.CompilerParams(use_tc_tiling_on_sc=False),
        scratch_types=[
            pltpu.VMEM((2, K2), jnp.int32),
            pltpu.VMEM((2, K2, CP), jnp.float32),
            pltpu.SemaphoreType.DMA,
            pltpu.SemaphoreType.DMA,
        ],
    )
    return f(ssm_pad, se)


def _dense_body(ml_ref, mr_ref, cnt_ref, x_ref, batch_ref,
                wl1_ref, wr1_ref, wla_ref, wra_ref, bl1_ref, bla_ref,
                ssm_ref, xp_ref):
    i = pl.program_id(0)
    cnt = jnp.maximum(cnt_ref[...], 1.0)
    a = jnp.concatenate([ml_ref[...], mr_ref[...]], axis=1) / cnt
    x = x_ref[...]
    z = jnp.dot(a, wl1_ref[...], preferred_element_type=jnp.float32)
    z += jnp.dot(x, wr1_ref[...], preferred_element_type=jnp.float32)
    z += bl1_ref[...]
    z = jnp.maximum(z, 0.0)
    s = jnp.dot(a, wla_ref[...], preferred_element_type=jnp.float32)
    s += jnp.dot(x, wra_ref[...], preferred_element_type=jnp.float32)
    s += bla_ref[...]
    ids = lax.broadcasted_iota(jnp.int32, (1, CP), 1)
    s = jnp.where(ids < C, s, -1e30)
    m = jnp.max(s, axis=1, keepdims=True)
    e = jnp.exp(s - m)
    ssm = e / jnp.sum(e, axis=1, keepdims=True)
    batchf = batch_ref[...].astype(jnp.float32)
    ssm_ref[...] = jnp.where(ids == C, batchf, ssm)

    sc8 = ssm[:, :C]
    gid64 = lax.broadcasted_iota(jnp.int32, (1, B * C), 1) // C
    oh64 = (batch_ref[...] == gid64).astype(jnp.float32)
    p = jnp.concatenate([sc8] * B, axis=1) * oh64
    xp = lax.dot_general(p, z, (((0,), (0,)), ((), ())),
                         preferred_element_type=jnp.float32)

    @pl.when(i == 0)
    def _():
        xp_ref[...] = xp

    @pl.when(i != 0)
    def _():
        xp_ref[...] += xp


def _dense_stage(msgl, msgr, cnt2d, x, batch2d, Wl1T, Wr1T, WlaT, WraT, bl1, bla):
    grid = (N // TN,)
    return pl.pallas_call(
        _dense_body,
        grid=grid,
        in_specs=[
            pl.BlockSpec((TN, H), lambda i: (i, 0)),
            pl.BlockSpec((TN, H), lambda i: (i, 0)),
            pl.BlockSpec((TN, 1), lambda i: (i, 0)),
            pl.BlockSpec((TN, D), lambda i: (i, 0)),
            pl.BlockSpec((TN, 1), lambda i: (i, 0)),
            pl.BlockSpec((D, D), lambda i: (0, 0)),
            pl.BlockSpec((D, D), lambda i: (0, 0)),
            pl.BlockSpec((D, CP), lambda i: (0, 0)),
            pl.BlockSpec((D, CP), lambda i: (0, 0)),
            pl.BlockSpec((1, D), lambda i: (0, 0)),
            pl.BlockSpec((1, CP), lambda i: (0, 0)),
        ],
        out_specs=[
            pl.BlockSpec((TN, CP), lambda i: (i, 0)),
            pl.BlockSpec((B * C, D), lambda i: (0, 0)),
        ],
        out_shape=[
            jax.ShapeDtypeStruct((N, CP), jnp.float32),
            jax.ShapeDtypeStruct((B * C, D), jnp.float32),
        ],
        interpret=_I,
    )(msgl, msgr, cnt2d, x, batch2d, Wl1T, Wr1T, WlaT, WraT, bl1, bla)


def _adj_head_body(u_ref, v_ref, xp_ref,
                   wl2_ref, wr2_ref, bl2_ref, w3_ref, b3_ref, w4_ref, b4_ref,
                   o_ref, a_acc):
    i = pl.program_id(0)
    gidf = (lax.broadcasted_iota(jnp.int32, (1, B * C), 1) // C
            ).astype(jnp.float32)
    up = u_ref[...]
    vp = v_ref[...]
    a = jnp.zeros((B * C, C), jnp.float32)
    for k in range(PK):
        u_k = up[:, k * CP:k * CP + C]
        bs_k = up[:, k * CP + C:k * CP + C + 1]
        v_k = vp[:, k * CP:k * CP + C]
        bd_k = vp[:, k * CP + C:k * CP + C + 1]
        m_k = ((bs_k == gidf) & (bd_k == gidf)).astype(jnp.float32)
        p_k = jnp.concatenate([u_k] * B, axis=1) * m_k
        a += lax.dot_general(p_k, v_k, (((0,), (0,)), ((), ())),
                             preferred_element_type=jnp.float32)

    @pl.when(i == 0)
    def _():
        a_acc[...] = a

    @pl.when(i != 0)
    def _():
        a_acc[...] += a

    @pl.when(i == pl.num_programs(0) - 1)
    def _():
        xp = xp_ref[...]
        ones = jnp.ones((C, 1), jnp.float32)
        parts = []
        for b in range(B):
            ab = a_acc[b * C:(b + 1) * C, :]
            mf = (ab != 0.0).astype(jnp.float32)
            c2 = lax.dot_general(mf, ones, (((0,), (0,)), ((), ())),
                                 preferred_element_type=jnp.float32)
            c2 = jnp.maximum(c2, 1.0)
            xb = xp[b * C:(b + 1) * C, :]
            ag = lax.dot_general(mf, xb, (((0,), (0,)), ((), ())),
                                 preferred_element_type=jnp.float32)
            parts.append(ag / c2)
        aggr2 = jnp.concatenate(parts, axis=0)
        zp = jnp.dot(aggr2, wl2_ref[...], preferred_element_type=jnp.float32)
        zp += jnp.dot(xp, wr2_ref[...], preferred_element_type=jnp.float32)
        zp += bl2_ref[...]
        zp = jnp.maximum(zp, 0.0)
        r64 = lax.broadcasted_iota(jnp.int32, (B * C, B * C), 0)
        c64 = lax.broadcasted_iota(jnp.int32, (B * C, B * C), 1)
        perm = ((r64 % B) * C + r64 // B == c64).astype(jnp.float32)
        zp_cm = jnp.dot(perm, zp, preferred_element_type=jnp.float32)
        ge = jnp.concatenate([zp_cm[c * B:(c + 1) * B, :] for c in range(C)],
                             axis=1)
        h = jnp.dot(ge, w3_ref[...], preferred_element_type=jnp.float32)
        h = jnp.maximum(h + b3_ref[...], 0.0)
        o = jnp.dot(h, w4_ref[...], preferred_element_type=jnp.float32)
        o_ref[...] = o + b4_ref[...]


def _adj_head_stage(uv, x_pool, Wl2T, Wr2T, bl2, W3T, b3, W4T, b4):
    grid = (E // TE,)
    tep = TE // PK
    return pl.pallas_call(
        _adj_head_body,
        grid=grid,
        in_specs=[
            pl.BlockSpec((tep, 128), lambda i: (i, 0)),
            pl.BlockSpec((tep, 128), lambda i: (E // TE + i, 0)),
            pl.BlockSpec((B * C, D), lambda i: (0, 0)),
            pl.BlockSpec((D, D), lambda i: (0, 0)),
            pl.BlockSpec((D, D), lambda i: (0, 0)),
            pl.BlockSpec((1, D), lambda i: (0, 0)),
            pl.BlockSpec((C * D, D), lambda i: (0, 0)),
            pl.BlockSpec((1, D), lambda i: (0, 0)),
            pl.BlockSpec((D, 1), lambda i: (0, 0)),
            pl.BlockSpec((1, 1), lambda i: (0, 0)),
        ],
        out_specs=pl.BlockSpec((B, 1), lambda i: (0, 0)),
        out_shape=jax.ShapeDtypeStruct((B, 1), jnp.float32),
        scratch_shapes=[pltpu.VMEM((B * C, C), jnp.float32)],
        interpret=_I,
    )(uv, uv, x_pool, Wl2T, Wr2T, bl2, W3T, b3, W4T, b4)


def kernel(x, edge_index, batch, Wl1, bl1, Wr1, Wla, bla, Wra, Wl2, bl2, Wr2, W3, b3, W4, b4):
    src = edge_index[0]
    dst = edge_index[1]

    msgl, msgr, cnt2d = _sc_scatter_stage(x, src, dst)

    WlaTp = jnp.pad(Wla.T, ((0, 0), (0, CP - C)))
    WraTp = jnp.pad(Wra.T, ((0, 0), (0, CP - C)))
    blap = jnp.pad(bla.reshape(1, C), ((0, 0), (0, CP - C)))
    ssm, x_pool = _dense_stage(
        msgl, msgr, cnt2d, x, batch.reshape(N, 1),
        Wl1.T, Wr1.T, WlaTp, WraTp,
        bl1.reshape(1, D), blap)

    uv = _sc_gather_stage(ssm, src, dst).reshape(2 * E // PK, 128)

    o = _adj_head_stage(uv, x_pool, Wl2.T, Wr2.T, bl2.reshape(1, D),
                        W3.T, b3.reshape(1, D), W4.T, b4.reshape(1, 1))
    return o.reshape(B)


# speedup vs baseline: 8.9992x; 1.0036x over previous
import functools

import jax
import jax.numpy as jnp
from jax import lax
from jax.experimental import pallas as pl
from jax.experimental.pallas import tpu as pltpu
from jax.experimental.pallas import tpu_sc as plsc

N = 10000
E = 160000
B = 8
D = 256
H = 128
Q = 64
C = 8
CP = 16
CW = 8
PK = 128 // CP

TN = 1000
TE = 8000

NC = 2
NS = 16
EPS = E // NS
CHUNK = 400
K2 = 2000
RW = 1000

_I = False


def _sc_mesh():
    return plsc.VectorSubcoreMesh(core_axis_name="c", subcore_axis_name="s",
                                  num_cores=NC, num_subcores=NS)


def _sc_pass(xh_hbm, idx2_hbm, dst_hbm, idx_s, idx_d, rows_v, acc, acc_cnt,
             ones_v, sems, core, sub, do_cnt):
    table = xh_hbm
    base = sub * EPS
    nch = EPS // CHUNK

    pltpu.sync_copy(idx2_hbm.at[pl.ds(core * E + base, CHUNK)], idx_s.at[0])
    pltpu.sync_copy(dst_hbm.at[pl.ds(base, CHUNK)], idx_d.at[0])
    descs = [None, None]
    descs[0] = pltpu.async_copy(table.at[idx_s.at[0]], rows_v.at[0], sems[0])
    for j in range(nch):
        b = j % 2
        nb = (j + 1) % 2
        if j + 1 < nch:
            off = base + (j + 1) * CHUNK
            pltpu.sync_copy(idx2_hbm.at[pl.ds(core * E + off, CHUNK)],
                            idx_s.at[nb])
            pltpu.sync_copy(dst_hbm.at[pl.ds(off, CHUNK)], idx_d.at[nb])
            descs[nb] = pltpu.async_copy(table.at[idx_s.at[nb]],
                                         rows_v.at[nb], sems[nb])
        descs[b].wait()
        pltpu.sync_copy(rows_v.at[b], acc.at[idx_d.at[b]], add=True)
        if do_cnt:
            @pl.when(core == 0)
            def _():
                pltpu.sync_copy(ones_v, acc_cnt.at[idx_d.at[b]], add=True)


def _sc_scatter_body(xl_hbm, xr_hbm, idx2_hbm, dst_hbm, zeros_hbm,
                     zcnt_hbm, ones_hbm, msgl_hbm, msgr_hbm, cnt_hbm,
                     idx_s, idx_d, rows_v, ones_v, acc, acc_cnt, sem0, sem1):
    core = lax.axis_index("c")
    sub = lax.axis_index("s")

    r0 = sub * RW

    @pl.when(sub < N // RW)
    def _():
        pltpu.sync_copy(zeros_hbm.at[pl.ds(r0, RW), :], acc.at[pl.ds(r0, RW), :])

    @pl.when((core == 0) & (sub == 0))
    def _():
        pltpu.sync_copy(zcnt_hbm, acc_cnt)

    @pl.when(core == 0)
    def _():
        pltpu.sync_copy(ones_hbm, ones_v)

    plsc.subcore_barrier()

    _sc_pass(xl_hbm, idx2_hbm, dst_hbm, idx_s, idx_d, rows_v, acc, acc_cnt,
             ones_v, (sem0, sem1), core, sub, True)

    plsc.subcore_barrier()

    @pl.when(sub < N // RW)
    def _():
        pltpu.sync_copy(acc.at[pl.ds(r0, RW), :],
                        msgl_hbm.at[pl.ds(r0, RW), pl.ds(core * Q, Q)])
        pltpu.sync_copy(zeros_hbm.at[pl.ds(r0, RW), :], acc.at[pl.ds(r0, RW), :])

    @pl.when((core == 0) & (sub == 0))
    def _():
        pltpu.sync_copy(acc_cnt, cnt_hbm)

    plsc.subcore_barrier()

    _sc_pass(xr_hbm, idx2_hbm, dst_hbm, idx_s, idx_d, rows_v, acc, acc_cnt,
             ones_v, (sem0, sem1), core, sub, False)

    plsc.subcore_barrier()

    @pl.when(sub < N // RW)
    def _():
        pltpu.sync_copy(acc.at[pl.ds(r0, RW), :],
                        msgr_hbm.at[pl.ds(r0, RW), pl.ds(core * Q, Q)])


def _sc_scatter_stage(x, src, dst):
    zeros = jnp.zeros((N, Q), jnp.float32)
    zcnt = jnp.zeros((N, CW), jnp.float32)
    ones = jnp.ones((CHUNK, CW), jnp.float32)
    idx2 = jnp.concatenate([2 * src, 2 * src + 1])
    msg_t = jax.ShapeDtypeStruct((N, H), jnp.float32)
    f = pl.kernel(
        _sc_scatter_body,
        out_type=[msg_t, msg_t, jax.ShapeDtypeStruct((N, CW), jnp.float32)],
        mesh=_sc_mesh(),
        compiler_params=pltpu.CompilerParams(use_tc_tiling_on_sc=False),
        scratch_types=[
            pltpu.VMEM((2, CHUNK), jnp.int32),
            pltpu.VMEM((2, CHUNK), jnp.int32),
            pltpu.VMEM((2, CHUNK, Q), jnp.float32),
            pltpu.VMEM((CHUNK, CW), jnp.float32),
            pltpu.VMEM_SHARED((N, Q), jnp.float32),
            pltpu.VMEM_SHARED((N, CW), jnp.float32),
            pltpu.SemaphoreType.DMA,
            pltpu.SemaphoreType.DMA,
        ],
    )
    xab1 = x[:, :H].reshape(2 * N, Q)
    xab2 = x[:, H:].reshape(2 * N, Q)
    msgl, msgr, cnt16 = f(xab1, xab2, idx2, dst, zeros, zcnt, ones)
    return msgl, msgr, cnt16[:, :1]


def _sc_gather_body(ssm_hbm, se_hbm, uv_hbm, idx_v, rows_v, sem0, sem1):
    core = lax.axis_index("c")
    sub = lax.axis_index("s")
    base = core * E + sub * EPS
    nch = EPS // K2
    sems = (sem0, sem1)
    pltpu.sync_copy(se_hbm.at[pl.ds(base, K2)], idx_v.at[0])
    descs = [None, None]
    descs[0] = pltpu.async_copy(ssm_hbm.at[idx_v.at[0]], rows_v.at[0], sems[0])
    for j in range(nch):
        b = j % 2
        nb = (j + 1) % 2
        if j + 1 < nch:
            off = base + (j + 1) * K2
            pltpu.sync_copy(se_hbm.at[pl.ds(off, K2)], idx_v.at[nb])
            descs[nb] = pltpu.async_copy(ssm_hbm.at[idx_v.at[nb]],
                                         rows_v.at[nb], sems[nb])
        descs[b].wait()
        pltpu.sync_copy(rows_v.at[b], uv_hbm.at[pl.ds(base + j * K2, K2), :])


def _sc_gather_stage(ssm_pad, src, dst):
    se = jnp.concatenate([src, dst])
    f = pl.kernel(
        _sc_gather_body,
        out_type=jax.ShapeDtypeStruct((2 * E, CP), jnp.float32),
        mesh=_sc_mesh(),
        compiler_params=pltpu.CompilerParams(use_tc_tiling_on_sc=False),
        scratch_types=[
            pltpu.VMEM((2, K2), jnp.int32),
            pltpu.VMEM((2, K2, CP), jnp.float32),
            pltpu.SemaphoreType.DMA,
            pltpu.SemaphoreType.DMA,
        ],
    )
    return f(ssm_pad, se)


def _dense_body(ml_ref, mr_ref, cnt_ref, x_ref, batch_ref,
                wl1_ref, wr1_ref, wla_ref, wra_ref, bl1_ref, bla_ref,
                ssm_ref, xp_ref):
    i = pl.program_id(0)
    cnt = jnp.maximum(cnt_ref[...], 1.0)
    a = jnp.concatenate([ml_ref[...], mr_ref[...]], axis=1) / cnt
    x = x_ref[...]
    z = jnp.dot(a, wl1_ref[...], preferred_element_type=jnp.float32)
    z += jnp.dot(x, wr1_ref[...], preferred_element_type=jnp.float32)
    z += bl1_ref[...]
    z = jnp.maximum(z, 0.0)
    s = jnp.dot(a, wla_ref[...], preferred_element_type=jnp.float32)
    s += jnp.dot(x, wra_ref[...], preferred_element_type=jnp.float32)
    s += bla_ref[...]
    ids = lax.broadcasted_iota(jnp.int32, (1, CP), 1)
    s = jnp.where(ids < C, s, -1e30)
    m = jnp.max(s, axis=1, keepdims=True)
    e = jnp.exp(s - m)
    ssm = e / jnp.sum(e, axis=1, keepdims=True)
    batchf = batch_ref[...].astype(jnp.float32)
    ssm_ref[...] = jnp.where(ids == C, batchf, ssm)

    sc8 = ssm[:, :C]
    gid64 = lax.broadcasted_iota(jnp.int32, (1, B * C), 1) // C
    oh64 = (batch_ref[...] == gid64).astype(jnp.float32)
    p = jnp.concatenate([sc8] * B, axis=1) * oh64
    xp = lax.dot_general(p, z, (((0,), (0,)), ((), ())),
                         preferred_element_type=jnp.float32)

    @pl.when(i == 0)
    def _():
        xp_ref[...] = xp

    @pl.when(i != 0)
    def _():
        xp_ref[...] += xp


def _dense_stage(msgl, msgr, cnt2d, x, batch2d, Wl1T, Wr1T, WlaT, WraT, bl1, bla):
    grid = (N // TN,)
    return pl.pallas_call(
        _dense_body,
        grid=grid,
        in_specs=[
            pl.BlockSpec((TN, H), lambda i: (i, 0)),
            pl.BlockSpec((TN, H), lambda i: (i, 0)),
            pl.BlockSpec((TN, 1), lambda i: (i, 0)),
            pl.BlockSpec((TN, D), lambda i: (i, 0)),
            pl.BlockSpec((TN, 1), lambda i: (i, 0)),
            pl.BlockSpec((D, D), lambda i: (0, 0)),
            pl.BlockSpec((D, D), lambda i: (0, 0)),
            pl.BlockSpec((D, CP), lambda i: (0, 0)),
            pl.BlockSpec((D, CP), lambda i: (0, 0)),
            pl.BlockSpec((1, D), lambda i: (0, 0)),
            pl.BlockSpec((1, CP), lambda i: (0, 0)),
        ],
        out_specs=[
            pl.BlockSpec((TN, CP), lambda i: (i, 0)),
            pl.BlockSpec((B * C, D), lambda i: (0, 0)),
        ],
        out_shape=[
            jax.ShapeDtypeStruct((N, CP), jnp.float32),
            jax.ShapeDtypeStruct((B * C, D), jnp.float32),
        ],
        interpret=_I,
    )(msgl, msgr, cnt2d, x, batch2d, Wl1T, Wr1T, WlaT, WraT, bl1, bla)


def _adj_head_body(u_ref, v_ref, xp_ref,
                   wl2_ref, wr2_ref, bl2_ref, w3_ref, b3_ref, w4_ref, b4_ref,
                   o_ref, a_acc):
    i = pl.program_id(0)
    gidf = (lax.broadcasted_iota(jnp.int32, (1, B * C), 1) // C
            ).astype(jnp.float32)
    up = u_ref[...]
    vp = v_ref[...]
    ps = []
    vs = []
    for k in range(PK):
        u_k = up[:, k * CP:k * CP + C]
        bs_k = up[:, k * CP + C:k * CP + C + 1]
        v_k = vp[:, k * CP:k * CP + C]
        bd_k = vp[:, k * CP + C:k * CP + C + 1]
        m_k = ((bs_k == gidf) & (bd_k == gidf)).astype(jnp.float32)
        ps.append(jnp.concatenate([u_k] * B, axis=1) * m_k)
        vs.append(v_k)
    a = lax.dot_general(jnp.concatenate(ps, axis=0), jnp.concatenate(vs, axis=0),
                        (((0,), (0,)), ((), ())),
                        preferred_element_type=jnp.float32)

    @pl.when(i == 0)
    def _():
        a_acc[...] = a

    @pl.when(i != 0)
    def _():
        a_acc[...] += a

    @pl.when(i == pl.num_programs(0) - 1)
    def _():
        xp = xp_ref[...]
        ones = jnp.ones((C, 1), jnp.float32)
        parts = []
        for b in range(B):
            ab = a_acc[b * C:(b + 1) * C, :]
            mf = (ab != 0.0).astype(jnp.float32)
            c2 = lax.dot_general(mf, ones, (((0,), (0,)), ((), ())),
                                 preferred_element_type=jnp.float32)
            c2 = jnp.maximum(c2, 1.0)
            xb = xp[b * C:(b + 1) * C, :]
            ag = lax.dot_general(mf, xb, (((0,), (0,)), ((), ())),
                                 preferred_element_type=jnp.float32)
            parts.append(ag / c2)
        aggr2 = jnp.concatenate(parts, axis=0)
        zp = jnp.dot(aggr2, wl2_ref[...], preferred_element_type=jnp.float32)
        zp += jnp.dot(xp, wr2_ref[...], preferred_element_type=jnp.float32)
        zp += bl2_ref[...]
        zp = jnp.maximum(zp, 0.0)
        r64 = lax.broadcasted_iota(jnp.int32, (B * C, B * C), 0)
        c64 = lax.broadcasted_iota(jnp.int32, (B * C, B * C), 1)
        perm = ((r64 % B) * C + r64 // B == c64).astype(jnp.float32)
        zp_cm = jnp.dot(perm, zp, preferred_element_type=jnp.float32)
        ge = jnp.concatenate([zp_cm[c * B:(c + 1) * B, :] for c in range(C)],
                             axis=1)
        h = jnp.dot(ge, w3_ref[...], preferred_element_type=jnp.float32)
        h = jnp.maximum(h + b3_ref[...], 0.0)
        o = jnp.dot(h, w4_ref[...], preferred_element_type=jnp.float32)
        o_ref[...] = o + b4_ref[...]


def _adj_head_stage(uv, x_pool, Wl2T, Wr2T, bl2, W3T, b3, W4T, b4):
    grid = (E // TE,)
    tep = TE // PK
    return pl.pallas_call(
        _adj_head_body,
        grid=grid,
        in_specs=[
            pl.BlockSpec((tep, 128), lambda i: (i, 0)),
            pl.BlockSpec((tep, 128), lambda i: (E // TE + i, 0)),
            pl.BlockSpec((B * C, D), lambda i: (0, 0)),
            pl.BlockSpec((D, D), lambda i: (0, 0)),
            pl.BlockSpec((D, D), lambda i: (0, 0)),
            pl.BlockSpec((1, D), lambda i: (0, 0)),
            pl.BlockSpec((C * D, D), lambda i: (0, 0)),
            pl.BlockSpec((1, D), lambda i: (0, 0)),
            pl.BlockSpec((D, 1), lambda i: (0, 0)),
            pl.BlockSpec((1, 1), lambda i: (0, 0)),
        ],
        out_specs=pl.BlockSpec((B, 1), lambda i: (0, 0)),
        out_shape=jax.ShapeDtypeStruct((B, 1), jnp.float32),
        scratch_shapes=[pltpu.VMEM((B * C, C), jnp.float32)],
        interpret=_I,
    )(uv, uv, x_pool, Wl2T, Wr2T, bl2, W3T, b3, W4T, b4)


def kernel(x, edge_index, batch, Wl1, bl1, Wr1, Wla, bla, Wra, Wl2, bl2, Wr2, W3, b3, W4, b4):
    src = edge_index[0]
    dst = edge_index[1]

    msgl, msgr, cnt2d = _sc_scatter_stage(x, src, dst)

    WlaTp = jnp.pad(Wla.T, ((0, 0), (0, CP - C)))
    WraTp = jnp.pad(Wra.T, ((0, 0), (0, CP - C)))
    blap = jnp.pad(bla.reshape(1, C), ((0, 0), (0, CP - C)))
    ssm, x_pool = _dense_stage(
        msgl, msgr, cnt2d, x, batch.reshape(N, 1),
        Wl1.T, Wr1.T, WlaTp, WraTp,
        bl1.reshape(1, D), blap)

    uv = _sc_gather_stage(ssm, src, dst).reshape(2 * E // PK, 128)

    o = _adj_head_stage(uv, x_pool, Wl2.T, Wr2.T, bl2.reshape(1, D),
                        W3.T, b3.reshape(1, D), W4.T, b4.reshape(1, 1))
    return o.reshape(B)


# speedup vs baseline: 8.9997x; 1.0001x over previous
import jax
import jax.numpy as jnp
from jax import lax
from jax.experimental import pallas as pl
from jax.experimental.pallas import tpu as pltpu
from jax.experimental.pallas import tpu_sc as plsc

N = 10000
E = 160000
B = 8
D = 256
H = 128
Q = 64
C = 8
CP = 16
CW = 8
PK = 128 // CP

TN = 1000
TE = 8000

NC = 2
NS = 16
EPS = E // NS
CHUNK = 400
K2 = 2000
RW = 1000

def _sc_mesh():
    return plsc.VectorSubcoreMesh(core_axis_name="c", subcore_axis_name="s",
                                  num_cores=NC, num_subcores=NS)


def _sc_pass(xh_hbm, idx2_hbm, dst_hbm, idx_s, idx_d, rows_v, acc, acc_cnt,
             ones_v, sems, core, sub, do_cnt):
    table = xh_hbm
    base = sub * EPS
    nch = EPS // CHUNK

    pltpu.sync_copy(idx2_hbm.at[pl.ds(core * E + base, CHUNK)], idx_s.at[0])
    pltpu.sync_copy(dst_hbm.at[pl.ds(base, CHUNK)], idx_d.at[0])
    descs = [None, None]
    descs[0] = pltpu.async_copy(table.at[idx_s.at[0]], rows_v.at[0], sems[0])
    for j in range(nch):
        b = j % 2
        nb = (j + 1) % 2
        if j + 1 < nch:
            off = base + (j + 1) * CHUNK
            pltpu.sync_copy(idx2_hbm.at[pl.ds(core * E + off, CHUNK)],
                            idx_s.at[nb])
            pltpu.sync_copy(dst_hbm.at[pl.ds(off, CHUNK)], idx_d.at[nb])
            descs[nb] = pltpu.async_copy(table.at[idx_s.at[nb]],
                                         rows_v.at[nb], sems[nb])
        descs[b].wait()
        pltpu.sync_copy(rows_v.at[b], acc.at[idx_d.at[b]], add=True)
        if do_cnt:
            @pl.when(core == 0)
            def _():
                pltpu.sync_copy(ones_v, acc_cnt.at[idx_d.at[b]], add=True)


def _sc_scatter_body(xl_hbm, xr_hbm, idx2_hbm, dst_hbm, zeros_hbm,
                     zcnt_hbm, ones_hbm, msgl_hbm, msgr_hbm, cnt_hbm,
                     idx_s, idx_d, rows_v, ones_v, acc, acc_cnt, sem0, sem1):
    core = lax.axis_index("c")
    sub = lax.axis_index("s")

    r0 = sub * RW

    @pl.when(sub < N // RW)
    def _():
        pltpu.sync_copy(zeros_hbm.at[pl.ds(r0, RW), :], acc.at[pl.ds(r0, RW), :])

    @pl.when((core == 0) & (sub == 0))
    def _():
        pltpu.sync_copy(zcnt_hbm, acc_cnt)

    @pl.when(core == 0)
    def _():
        pltpu.sync_copy(ones_hbm, ones_v)

    plsc.subcore_barrier()

    _sc_pass(xl_hbm, idx2_hbm, dst_hbm, idx_s, idx_d, rows_v, acc, acc_cnt,
             ones_v, (sem0, sem1), core, sub, True)

    plsc.subcore_barrier()

    @pl.when(sub < N // RW)
    def _():
        pltpu.sync_copy(acc.at[pl.ds(r0, RW), :],
                        msgl_hbm.at[pl.ds(r0, RW), pl.ds(core * Q, Q)])
        pltpu.sync_copy(zeros_hbm.at[pl.ds(r0, RW), :], acc.at[pl.ds(r0, RW), :])

    @pl.when((core == 0) & (sub == 0))
    def _():
        pltpu.sync_copy(acc_cnt, cnt_hbm)

    plsc.subcore_barrier()

    _sc_pass(xr_hbm, idx2_hbm, dst_hbm, idx_s, idx_d, rows_v, acc, acc_cnt,
             ones_v, (sem0, sem1), core, sub, False)

    plsc.subcore_barrier()

    @pl.when(sub < N // RW)
    def _():
        pltpu.sync_copy(acc.at[pl.ds(r0, RW), :],
                        msgr_hbm.at[pl.ds(r0, RW), pl.ds(core * Q, Q)])


def _sc_scatter_stage(x, src, dst):
    zeros = jnp.zeros((N, Q), jnp.float32)
    zcnt = jnp.zeros((N, CW), jnp.float32)
    ones = jnp.ones((CHUNK, CW), jnp.float32)
    idx2 = jnp.concatenate([2 * src, 2 * src + 1])
    msg_t = jax.ShapeDtypeStruct((N, H), jnp.float32)
    f = pl.kernel(
        _sc_scatter_body,
        out_type=[msg_t, msg_t, jax.ShapeDtypeStruct((N, CW), jnp.float32)],
        mesh=_sc_mesh(),
        compiler_params=pltpu.CompilerParams(use_tc_tiling_on_sc=False),
        scratch_types=[
            pltpu.VMEM((2, CHUNK), jnp.int32),
            pltpu.VMEM((2, CHUNK), jnp.int32),
            pltpu.VMEM((2, CHUNK, Q), jnp.float32),
            pltpu.VMEM((CHUNK, CW), jnp.float32),
            pltpu.VMEM_SHARED((N, Q), jnp.float32),
            pltpu.VMEM_SHARED((N, CW), jnp.float32),
            pltpu.SemaphoreType.DMA,
            pltpu.SemaphoreType.DMA,
        ],
    )
    xab1 = x[:, :H].reshape(2 * N, Q)
    xab2 = x[:, H:].reshape(2 * N, Q)
    msgl, msgr, cnt16 = f(xab1, xab2, idx2, dst, zeros, zcnt, ones)
    return msgl, msgr, cnt16[:, :1]


def _sc_gather_body(ssm_hbm, se_hbm, uv_hbm, idx_v, rows_v, sem0, sem1):
    core = lax.axis_index("c")
    sub = lax.axis_index("s")
    base = core * E + sub * EPS
    nch = EPS // K2
    sems = (sem0, sem1)
    pltpu.sync_copy(se_hbm.at[pl.ds(base, K2)], idx_v.at[0])
    descs = [None, None]
    descs[0] = pltpu.async_copy(ssm_hbm.at[idx_v.at[0]], rows_v.at[0], sems[0])
    for j in range(nch):
        b = j % 2
        nb = (j + 1) % 2
        if j + 1 < nch:
            off = base + (j + 1) * K2
            pltpu.sync_copy(se_hbm.at[pl.ds(off, K2)], idx_v.at[nb])
            descs[nb] = pltpu.async_copy(ssm_hbm.at[idx_v.at[nb]],
                                         rows_v.at[nb], sems[nb])
        descs[b].wait()
        pltpu.sync_copy(rows_v.at[b], uv_hbm.at[pl.ds(base + j * K2, K2), :])


def _sc_gather_stage(ssm_pad, src, dst):
    se = jnp.concatenate([src, dst])
    f = pl.kernel(
        _sc_gather_body,
        out_type=jax.ShapeDtypeStruct((2 * E, CP), jnp.float32),
        mesh=_sc_mesh(),
        compiler_params=pltpu.CompilerParams(use_tc_tiling_on_sc=False),
        scratch_types=[
            pltpu.VMEM((2, K2), jnp.int32),
            pltpu.VMEM((2, K2, CP), jnp.float32),
            pltpu.SemaphoreType.DMA,
            pltpu.SemaphoreType.DMA,
        ],
    )
    return f(ssm_pad, se)


def _dense_body(ml_ref, mr_ref, cnt_ref, x_ref, batch_ref,
                wl1_ref, wr1_ref, wla_ref, wra_ref, bl1_ref, bla_ref,
                ssm_ref, xp_ref):
    i = pl.program_id(0)
    cnt = jnp.maximum(cnt_ref[...], 1.0)
    a = jnp.concatenate([ml_ref[...], mr_ref[...]], axis=1) / cnt
    x = x_ref[...]
    z = jnp.dot(a, wl1_ref[...], preferred_element_type=jnp.float32)
    z += jnp.dot(x, wr1_ref[...], preferred_element_type=jnp.float32)
    z += bl1_ref[...]
    z = jnp.maximum(z, 0.0)
    s = jnp.dot(a, wla_ref[...], preferred_element_type=jnp.float32)
    s += jnp.dot(x, wra_ref[...], preferred_element_type=jnp.float32)
    s += bla_ref[...]
    ids = lax.broadcasted_iota(jnp.int32, (1, CP), 1)
    s = jnp.where(ids < C, s, -1e30)
    m = jnp.max(s, axis=1, keepdims=True)
    e = jnp.exp(s - m)
    ssm = e / jnp.sum(e, axis=1, keepdims=True)
    batchf = batch_ref[...].astype(jnp.float32)
    ssm_ref[...] = jnp.where(ids == C, batchf, ssm)

    sc8 = ssm[:, :C]
    gid64 = lax.broadcasted_iota(jnp.int32, (1, B * C), 1) // C
    oh64 = (batch_ref[...] == gid64).astype(jnp.float32)
    p = jnp.concatenate([sc8] * B, axis=1) * oh64
    xp = lax.dot_general(p, z, (((0,), (0,)), ((), ())),
                         preferred_element_type=jnp.float32)

    @pl.when(i == 0)
    def _():
        xp_ref[...] = xp

    @pl.when(i != 0)
    def _():
        xp_ref[...] += xp


def _dense_stage(msgl, msgr, cnt2d, x, batch2d, Wl1T, Wr1T, WlaT, WraT, bl1, bla):
    grid = (N // TN,)
    return pl.pallas_call(
        _dense_body,
        grid=grid,
        in_specs=[
            pl.BlockSpec((TN, H), lambda i: (i, 0)),
            pl.BlockSpec((TN, H), lambda i: (i, 0)),
            pl.BlockSpec((TN, 1), lambda i: (i, 0)),
            pl.BlockSpec((TN, D), lambda i: (i, 0)),
            pl.BlockSpec((TN, 1), lambda i: (i, 0)),
            pl.BlockSpec((D, D), lambda i: (0, 0)),
            pl.BlockSpec((D, D), lambda i: (0, 0)),
            pl.BlockSpec((D, CP), lambda i: (0, 0)),
            pl.BlockSpec((D, CP), lambda i: (0, 0)),
            pl.BlockSpec((1, D), lambda i: (0, 0)),
            pl.BlockSpec((1, CP), lambda i: (0, 0)),
        ],
        out_specs=[
            pl.BlockSpec((TN, CP), lambda i: (i, 0)),
            pl.BlockSpec((B * C, D), lambda i: (0, 0)),
        ],
        out_shape=[
            jax.ShapeDtypeStruct((N, CP), jnp.float32),
            jax.ShapeDtypeStruct((B * C, D), jnp.float32),
        ],
    )(msgl, msgr, cnt2d, x, batch2d, Wl1T, Wr1T, WlaT, WraT, bl1, bla)


def _adj_head_body(u_ref, v_ref, xp_ref,
                   wl2_ref, wr2_ref, bl2_ref, w3_ref, b3_ref, w4_ref, b4_ref,
                   o_ref, a_acc):
    i = pl.program_id(0)
    gidf = (lax.broadcasted_iota(jnp.int32, (1, B * C), 1) // C
            ).astype(jnp.float32)
    up = u_ref[...]
    vp = v_ref[...]
    ps = []
    vs = []
    for k in range(PK):
        u_k = up[:, k * CP:k * CP + C]
        bs_k = up[:, k * CP + C:k * CP + C + 1]
        v_k = vp[:, k * CP:k * CP + C]
        bd_k = vp[:, k * CP + C:k * CP + C + 1]
        m_k = ((bs_k == gidf) & (bd_k == gidf)).astype(jnp.float32)
        ps.append(jnp.concatenate([u_k] * B, axis=1) * m_k)
        vs.append(v_k)
    a = lax.dot_general(jnp.concatenate(ps, axis=0), jnp.concatenate(vs, axis=0),
                        (((0,), (0,)), ((), ())),
                        preferred_element_type=jnp.float32)

    @pl.when(i == 0)
    def _():
        a_acc[...] = a

    @pl.when(i != 0)
    def _():
        a_acc[...] += a

    @pl.when(i == pl.num_programs(0) - 1)
    def _():
        xp = xp_ref[...]
        ones = jnp.ones((C, 1), jnp.float32)
        parts = []
        for b in range(B):
            ab = a_acc[b * C:(b + 1) * C, :]
            mf = (ab != 0.0).astype(jnp.float32)
            c2 = lax.dot_general(mf, ones, (((0,), (0,)), ((), ())),
                                 preferred_element_type=jnp.float32)
            c2 = jnp.maximum(c2, 1.0)
            xb = xp[b * C:(b + 1) * C, :]
            ag = lax.dot_general(mf, xb, (((0,), (0,)), ((), ())),
                                 preferred_element_type=jnp.float32)
            parts.append(ag / c2)
        aggr2 = jnp.concatenate(parts, axis=0)
        zp = jnp.dot(aggr2, wl2_ref[...], preferred_element_type=jnp.float32)
        zp += jnp.dot(xp, wr2_ref[...], preferred_element_type=jnp.float32)
        zp += bl2_ref[...]
        zp = jnp.maximum(zp, 0.0)
        r64 = lax.broadcasted_iota(jnp.int32, (B * C, B * C), 0)
        c64 = lax.broadcasted_iota(jnp.int32, (B * C, B * C), 1)
        perm = ((r64 % B) * C + r64 // B == c64).astype(jnp.float32)
        zp_cm = jnp.dot(perm, zp, preferred_element_type=jnp.float32)
        ge = jnp.concatenate([zp_cm[c * B:(c + 1) * B, :] for c in range(C)],
                             axis=1)
        h = jnp.dot(ge, w3_ref[...], preferred_element_type=jnp.float32)
        h = jnp.maximum(h + b3_ref[...], 0.0)
        o = jnp.dot(h, w4_ref[...], preferred_element_type=jnp.float32)
        o_ref[...] = o + b4_ref[...]


def _adj_head_stage(uv, x_pool, Wl2T, Wr2T, bl2, W3T, b3, W4T, b4):
    grid = (E // TE,)
    tep = TE // PK
    return pl.pallas_call(
        _adj_head_body,
        grid=grid,
        in_specs=[
            pl.BlockSpec((tep, 128), lambda i: (i, 0)),
            pl.BlockSpec((tep, 128), lambda i: (E // TE + i, 0)),
            pl.BlockSpec((B * C, D), lambda i: (0, 0)),
            pl.BlockSpec((D, D), lambda i: (0, 0)),
            pl.BlockSpec((D, D), lambda i: (0, 0)),
            pl.BlockSpec((1, D), lambda i: (0, 0)),
            pl.BlockSpec((C * D, D), lambda i: (0, 0)),
            pl.BlockSpec((1, D), lambda i: (0, 0)),
            pl.BlockSpec((D, 1), lambda i: (0, 0)),
            pl.BlockSpec((1, 1), lambda i: (0, 0)),
        ],
        out_specs=pl.BlockSpec((B, 1), lambda i: (0, 0)),
        out_shape=jax.ShapeDtypeStruct((B, 1), jnp.float32),
        scratch_shapes=[pltpu.VMEM((B * C, C), jnp.float32)],
    )(uv, uv, x_pool, Wl2T, Wr2T, bl2, W3T, b3, W4T, b4)


def kernel(x, edge_index, batch, Wl1, bl1, Wr1, Wla, bla, Wra, Wl2, bl2, Wr2, W3, b3, W4, b4):
    src = edge_index[0]
    dst = edge_index[1]

    msgl, msgr, cnt2d = _sc_scatter_stage(x, src, dst)

    WlaTp = jnp.pad(Wla.T, ((0, 0), (0, CP - C)))
    WraTp = jnp.pad(Wra.T, ((0, 0), (0, CP - C)))
    blap = jnp.pad(bla.reshape(1, C), ((0, 0), (0, CP - C)))
    ssm, x_pool = _dense_stage(
        msgl, msgr, cnt2d, x, batch.reshape(N, 1),
        Wl1.T, Wr1.T, WlaTp, WraTp,
        bl1.reshape(1, D), blap)

    uv = _sc_gather_stage(ssm, src, dst).reshape(2 * E // PK, 128)

    o = _adj_head_stage(uv, x_pool, Wl2.T, Wr2.T, bl2.reshape(1, D),
                        W3.T, b3.reshape(1, D), W4.T, b4.reshape(1, 1))
    return o.reshape(B)
